# baseline (speedup 1.0000x reference)
import functools

import jax
import jax.numpy as jnp
from jax.experimental import pallas as pl
from jax.experimental.pallas import tpu as pltpu

NUM_NEIGHBORS = 16
OUT_CHANNELS = 32
KP1 = NUM_NEIGHBORS + 1
R = 256
BIG = 3.0e38


def _knn_kernel(rows_ref, ptsT_ref, ptsP_ref, out_ref, scratch_ref):
    rows = rows_ref[0]
    ptsT = ptsT_ref[0]
    ptsP = ptsP_ref[0]
    n = ptsT.shape[1]
    sqr = jnp.sum(rows * rows, axis=1, keepdims=True)
    sqc = jnp.sum(ptsT * ptsT, axis=0, keepdims=True)
    dot = jnp.dot(rows, ptsT, preferred_element_type=jnp.float32)
    dist = (sqr + sqc) - 2.0 * dot
    iota = jax.lax.broadcasted_iota(jnp.int32, (R, n), 1)

    def body(k, d):
        minv = jnp.min(d, axis=1, keepdims=True)
        hit = d == minv
        idxv = jnp.min(jnp.where(hit, iota, n), axis=1, keepdims=True)
        sel = iota == idxv
        coords = jax.lax.dot_general(
            sel.astype(jnp.float32), ptsP,
            (((1,), (0,)), ((), ())),
            precision=jax.lax.Precision.HIGHEST,
            preferred_element_type=jnp.float32)
        scratch_ref[k] = coords
        return jnp.where(sel, BIG, d)

    jax.lax.fori_loop(0, KP1, body, dist, unroll=False)
    out_ref[0] = scratch_ref[...]


def _knn_coords(pts):
    Bb, Nn, _ = pts.shape
    pts_pad = jnp.pad(pts, ((0, 0), (0, 0), (0, 5)))
    ptsT = jnp.transpose(pts_pad, (0, 2, 1))
    nb = Nn // R
    return pl.pallas_call(
        _knn_kernel,
        grid=(Bb, nb),
        in_specs=[
            pl.BlockSpec((1, R, 8), lambda b, i: (b, i, 0)),
            pl.BlockSpec((1, 8, Nn), lambda b, i: (b, 0, 0)),
            pl.BlockSpec((1, Nn, 8), lambda b, i: (b, 0, 0)),
        ],
        out_specs=pl.BlockSpec((1, KP1, R, 8), lambda b, i: (b * nb + i, 0, 0, 0)),
        out_shape=jax.ShapeDtypeStruct((Bb * nb, KP1, R, 8), jnp.float32),
        scratch_shapes=[pltpu.VMEM((KP1, R, 8), jnp.float32)],
    )(pts_pad, ptsT, pts_pad)


def _bn_scale_shift(s, ss, cnt, g, b, eps=1e-5):
    mean = s / cnt
    var = ss / cnt - mean * mean
    scale = g / jnp.sqrt(var + eps)
    shift = b - mean * scale
    return scale.reshape(1, -1), shift.reshape(1, -1)


def _acc_init(i0_ref, i1_ref):
    @pl.when(pl.program_id(0) == 0)
    def _():
        i0_ref[...] = jnp.zeros_like(i0_ref)
        i1_ref[...] = jnp.zeros_like(i1_ref)


def _feat_y1_kernel(c_ref, w_ref, y1_ref, s_ref, ss_ref):
    c = c_ref[0][:, :, :3]
    ab = c[:1]
    rel = c[1:] - ab
    d = jnp.sqrt(jnp.sum(rel * rel, axis=2, keepdims=True) + 1e-8)
    feat = jnp.concatenate(
        (jnp.broadcast_to(ab, rel.shape), rel, d,
         jnp.zeros_like(d)), axis=2)
    y1 = jnp.dot(feat.reshape(16 * R, 8), w_ref[...],
                 preferred_element_type=jnp.float32)
    y1_ref[0] = y1.reshape(16, R, 16)
    _acc_init(s_ref, ss_ref)
    s_ref[...] += jnp.sum(y1, axis=0, keepdims=True)
    ss_ref[...] += jnp.sum(y1 * y1, axis=0, keepdims=True)


def _y2_kernel(y1_ref, w_ref, sc_ref, sh_ref, y2_ref, s_ref, ss_ref):
    h1 = jnp.maximum(y1_ref[0].reshape(16 * R, 16) * sc_ref[...]
                     + sh_ref[...], 0.0)
    y2 = jnp.dot(h1, w_ref[...], preferred_element_type=jnp.float32)
    y2_ref[0] = y2.reshape(16, R, 32)
    _acc_init(s_ref, ss_ref)
    s_ref[...] += jnp.sum(y2, axis=0, keepdims=True)
    ss_ref[...] += jnp.sum(y2 * y2, axis=0, keepdims=True)


def _pool_lift_kernel(y2_ref, pts_ref, w2_ref, sc_ref, sh_ref,
                      pool_ref, y3_ref, s_ref, ss_ref):
    h2 = jnp.maximum(y2_ref[0] * sc_ref[...].reshape(1, 1, 32)
                     + sh_ref[...].reshape(1, 1, 32), 0.0)
    pool_ref[0] = jnp.max(h2, axis=0)
    y3 = jnp.dot(pts_ref[0], w2_ref[...],
                 preferred_element_type=jnp.float32)
    y3_ref[0] = y3
    _acc_init(s_ref, ss_ref)
    s_ref[...] += jnp.sum(y3, axis=0, keepdims=True)
    ss_ref[...] += jnp.sum(y3 * y3, axis=0, keepdims=True)


def _y4_kernel(y3_ref, pool_ref, w3_ref, sc_ref, sh_ref,
               y4_ref, s_ref, ss_ref):
    lifted = jnp.maximum(y3_ref[0] * sc_ref[...] + sh_ref[...], 0.0)
    xcat = jnp.concatenate((lifted, pool_ref[0]), axis=1)
    y4 = jnp.dot(xcat, w3_ref[...], preferred_element_type=jnp.float32)
    y4_ref[0] = y4
    _acc_init(s_ref, ss_ref)
    s_ref[...] += jnp.sum(y4, axis=0, keepdims=True)
    ss_ref[...] += jnp.sum(y4 * y4, axis=0, keepdims=True)


def _final_kernel(y4_ref, sc_ref, sh_ref, o_ref):
    o_ref[0] = jnp.maximum(y4_ref[0] * sc_ref[...] + sh_ref[...], 0.0)


def _full_spec(*shape):
    nd = len(shape)
    return pl.BlockSpec(shape, lambda m: (0,) * nd)


def _mblk_spec(*rest):
    return pl.BlockSpec((1,) + rest, lambda m: (m,) + (0,) * len(rest))


def kernel(pts, W1a, g1a, b1a, W1b, g1b, b1b, W2, g2, b2, W3, g3, b3):
    Bb, Nn, _ = pts.shape
    nb = Nn // R
    M = Bb * nb
    cnt_nbr = float(Bb * Nn * 16)
    cnt_pts = float(Bb * Nn)
    coords = _knn_coords(pts)
    pts_pad = jnp.pad(pts, ((0, 0), (0, 0), (0, 5))).reshape(M, R, 8)
    W1a_p = jnp.pad(W1a, ((0, 1), (0, 0)))
    W2_p = jnp.pad(W2, ((0, 5), (0, 0)))

    f32 = jnp.float32
    y1, s1, ss1 = pl.pallas_call(
        _feat_y1_kernel, grid=(M,),
        in_specs=[_mblk_spec(KP1, R, 8), _full_spec(8, 16)],
        out_specs=[_mblk_spec(16, R, 16), _full_spec(1, 16), _full_spec(1, 16)],
        out_shape=[jax.ShapeDtypeStruct((M, 16, R, 16), f32),
                   jax.ShapeDtypeStruct((1, 16), f32),
                   jax.ShapeDtypeStruct((1, 16), f32)],
    )(coords, W1a_p)
    sc1, sh1 = _bn_scale_shift(s1[0], ss1[0], cnt_nbr, g1a, b1a)

    y2, s2, ss2 = pl.pallas_call(
        _y2_kernel, grid=(M,),
        in_specs=[_mblk_spec(16, R, 16), _full_spec(16, 32),
                  _full_spec(1, 16), _full_spec(1, 16)],
        out_specs=[_mblk_spec(16, R, 32), _full_spec(1, 32), _full_spec(1, 32)],
        out_shape=[jax.ShapeDtypeStruct((M, 16, R, 32), f32),
                   jax.ShapeDtypeStruct((1, 32), f32),
                   jax.ShapeDtypeStruct((1, 32), f32)],
    )(y1, W1b, sc1, sh1)
    sc2, sh2 = _bn_scale_shift(s2[0], ss2[0], cnt_nbr, g1b, b1b)

    pooled, y3, s3, ss3 = pl.pallas_call(
        _pool_lift_kernel, grid=(M,),
        in_specs=[_mblk_spec(16, R, 32), _mblk_spec(R, 8),
                  _full_spec(8, 32), _full_spec(1, 32), _full_spec(1, 32)],
        out_specs=[_mblk_spec(R, 32), _mblk_spec(R, 32),
                   _full_spec(1, 32), _full_spec(1, 32)],
        out_shape=[jax.ShapeDtypeStruct((M, R, 32), f32),
                   jax.ShapeDtypeStruct((M, R, 32), f32),
                   jax.ShapeDtypeStruct((1, 32), f32),
                   jax.ShapeDtypeStruct((1, 32), f32)],
    )(y2, pts_pad, W2_p, sc2, sh2)
    sc3, sh3 = _bn_scale_shift(s3[0], ss3[0], cnt_pts, g2, b2)

    y4, s4, ss4 = pl.pallas_call(
        _y4_kernel, grid=(M,),
        in_specs=[_mblk_spec(R, 32), _mblk_spec(R, 32), _full_spec(64, 32),
                  _full_spec(1, 32), _full_spec(1, 32)],
        out_specs=[_mblk_spec(R, 32), _full_spec(1, 32), _full_spec(1, 32)],
        out_shape=[jax.ShapeDtypeStruct((M, R, 32), f32),
                   jax.ShapeDtypeStruct((1, 32), f32),
                   jax.ShapeDtypeStruct((1, 32), f32)],
    )(y3, pooled, W3, sc3, sh3)
    sc4, sh4 = _bn_scale_shift(s4[0], ss4[0], cnt_pts, g3, b3)

    out = pl.pallas_call(
        _final_kernel, grid=(M,),
        in_specs=[_mblk_spec(R, 32), _full_spec(1, 32), _full_spec(1, 32)],
        out_specs=_mblk_spec(R, 32),
        out_shape=jax.ShapeDtypeStruct((M, R, 32), f32),
    )(y4, sc4, sh4)
    return out.reshape(Bb, Nn, OUT_CHANNELS)


# speedup vs baseline: 1.7778x; 1.7778x over previous
import functools

import jax
import jax.numpy as jnp
from jax import lax
from jax.experimental import pallas as pl
from jax.experimental.pallas import tpu as pltpu
from jax.experimental.pallas import tpu_sc as plsc

NUM_NEIGHBORS = 16
OUT_CHANNELS = 32
KP1 = NUM_NEIGHBORS + 1
R = 256
BIG = 3.0e38


def _knn_kernel(rows_ref, ptsT_ref, ptsP_ref, out_ref, scratch_ref):
    rows = rows_ref[0]
    ptsT = ptsT_ref[0]
    ptsP = ptsP_ref[0]
    n = ptsT.shape[1]
    sqr = jnp.sum(rows * rows, axis=1, keepdims=True)
    sqc = jnp.sum(ptsT * ptsT, axis=0, keepdims=True)
    dot = jnp.dot(rows, ptsT, preferred_element_type=jnp.float32)
    dist = (sqr + sqc) - 2.0 * dot
    iota = jax.lax.broadcasted_iota(jnp.int32, (R, n), 1)

    def body(k, d):
        minv = jnp.min(d, axis=1, keepdims=True)
        hit = d == minv
        idxv = jnp.min(jnp.where(hit, iota, n), axis=1, keepdims=True)
        sel = iota == idxv
        coords = jax.lax.dot_general(
            sel.astype(jnp.float32), ptsP,
            (((1,), (0,)), ((), ())),
            precision=jax.lax.Precision.HIGHEST,
            preferred_element_type=jnp.float32)
        scratch_ref[k] = coords
        return jnp.where(sel, BIG, d)

    jax.lax.fori_loop(0, KP1, body, dist, unroll=False)
    out_ref[0] = scratch_ref[...]


def _knn_coords(pts):
    Bb, Nn, _ = pts.shape
    pts_pad = jnp.pad(pts, ((0, 0), (0, 0), (0, 5)))
    ptsT = jnp.transpose(pts_pad, (0, 2, 1))
    nb = Nn // R
    return pl.pallas_call(
        _knn_kernel,
        grid=(Bb, nb),
        in_specs=[
            pl.BlockSpec((1, R, 8), lambda b, i: (b, i, 0)),
            pl.BlockSpec((1, 8, Nn), lambda b, i: (b, 0, 0)),
            pl.BlockSpec((1, Nn, 8), lambda b, i: (b, 0, 0)),
        ],
        out_specs=pl.BlockSpec((1, KP1, R, 8), lambda b, i: (b * nb + i, 0, 0, 0)),
        out_shape=jax.ShapeDtypeStruct((Bb * nb, KP1, R, 8), jnp.float32),
        scratch_shapes=[pltpu.VMEM((KP1, R, 8), jnp.float32)],
    )(pts_pad, ptsT, pts_pad)


CAND = 128
MAXI = 2147483647


def _sc_knn_coords(pts):
    Bb, Nn, _ = pts.shape
    nb = Nn // R
    M = Bb * nb
    wpb = 32 // Bb
    bpw = nb // wpb
    ptsx = pts[:, :, 0]
    ptsy = pts[:, :, 1]
    ptsz = pts[:, :, 2]
    f32, i32 = jnp.float32, jnp.int32
    mesh = plsc.VectorSubcoreMesh(core_axis_name="c", subcore_axis_name="s")

    @functools.partial(
        pl.kernel,
        out_type=jax.ShapeDtypeStruct((M, KP1, R, 4), f32),
        mesh=mesh,
        compiler_params=pltpu.CompilerParams(
            needs_layout_passes=False, use_tc_tiling_on_sc=False),
        scratch_types=[
            pltpu.VMEM((Nn,), f32),
            pltpu.VMEM((Nn,), f32),
            pltpu.VMEM((Nn,), f32),
            pltpu.VMEM((Nn,), f32),
            pltpu.VMEM((Nn,), f32),
            pltpu.VMEM((Nn,), f32),
            pltpu.VMEM((Nn,), f32),
            pltpu.VMEM((Nn,), f32),
            pltpu.VMEM((32,), f32),
            pltpu.VMEM((CAND + 16,), f32),
            pltpu.VMEM((CAND + 16,), i32),
            pltpu.VMEM((32,), i32),
            pltpu.VMEM((KP1, R, 4), f32),
        ],
    )
    def sc_kernel(x_hbm, y_hbm, z_hbm, out_hbm,
                  xb, yb, zb, xr, yr, zr, sqb, db, gmb, cdb, cib, sib, ob):
        cid = lax.axis_index("c")
        sid = lax.axis_index("s")
        w = sid * 2 + cid
        b = w // wpb
        blk0 = (w % wpb) * bpw
        pltpu.sync_copy(x_hbm.at[b], xb)
        pltpu.sync_copy(y_hbm.at[b], yb)
        pltpu.sync_copy(z_hbm.at[b], zb)
        lanes = lax.iota(i32, 16)
        lane0 = lanes == 0
        big = jnp.full((16,), BIG, f32)

        def _rnd_bf16(x):
            u = plsc.bitcast(x, i32)
            u = (u + 0x7FFF + ((u >> 16) & 1)) & ~0xFFFF
            return plsc.bitcast(u, f32)

        def prep(j, _):
            sl = pl.ds(j * 16, 16)
            x = xb[sl]
            y = yb[sl]
            z = zb[sl]
            xr[sl] = _rnd_bf16(x)
            yr[sl] = _rnd_bf16(y)
            zr[sl] = _rnd_bf16(z)
            sqb[sl] = (x * x + y * y) + z * z
            return 0
        lax.fori_loop(0, Nn // 16, prep, 0)

        def per_query(i, blk_base):
            qi = blk_base + i
            qiv = jnp.full((16,), qi, i32)
            qx = plsc.load_gather(xr, [qiv])
            qy = plsc.load_gather(yr, [qiv])
            qz = plsc.load_gather(zr, [qiv])
            qsq = plsc.load_gather(sqb, [qiv])

            def gbody(g, _):
                def inner(jj, acc):
                    off = g * 128 + jj * 16
                    sl = pl.ds(off, 16)
                    dot = (qx * xr[sl] + qy * yr[sl]) + qz * zr[sl]
                    d = (qsq + sqb[sl]) - 2.0 * dot
                    db[sl] = d
                    return jnp.minimum(acc, d)
                acc = lax.fori_loop(0, 8, inner, big)
                plsc.store_scatter(gmb, [jnp.full((16,), g, i32)],
                                   jnp.full((16,), jnp.min(acc), f32),
                                   mask=lane0)
                return 0
            lax.fori_loop(0, 32, gbody, 0)

            def tbody(k, carry):
                g0, g1, _ = carry
                m = jnp.min(jnp.minimum(g0, g1))
                return (jnp.where(g0 == m, BIG, g0),
                        jnp.where(g1 == m, BIG, g1), m)
            _, _, T = lax.fori_loop(
                0, KP1, tbody,
                (gmb[pl.ds(0, 16)], gmb[pl.ds(16, 16)], 0.0))

            for v in range(CAND // 16 + 1):
                cdb[pl.ds(v * 16, 16)] = big
                cib[pl.ds(v * 16, 16)] = jnp.zeros((16,), i32)

            def cbody(j, off):
                d = db[pl.ds(j * 16, 16)]
                msk = d <= T
                plsc.store_compressed(cdb.at[pl.ds(off, 16)], d, mask=msk)
                plsc.store_compressed(cib.at[pl.ds(off, 16)],
                                      lanes + j * 16, mask=msk)
                cnt = jnp.max(plsc.all_reduce_population_count(msk))
                return jnp.minimum(off + cnt, CAND)
            lax.fori_loop(0, Nn // 16, cbody, 0)

            cds = tuple(cdb[pl.ds(v * 16, 16)] for v in range(CAND // 16))
            cis = tuple(cib[pl.ds(v * 16, 16)] for v in range(CAND // 16))
            sib[pl.ds(16, 16)] = jnp.zeros((16,), i32)

            def sbody(k, ds_):
                mm = ds_[0]
                for v in range(1, len(ds_)):
                    mm = jnp.minimum(mm, ds_[v])
                m = jnp.min(mm)
                ii = jnp.full((16,), MAXI, i32)
                for v in range(len(ds_)):
                    ii = jnp.minimum(
                        ii, jnp.where(ds_[v] == m, cis[v], MAXI))
                imin = jnp.min(ii)
                plsc.store_scatter(sib, [jnp.full((16,), k, i32)],
                                   jnp.full((16,), imin, i32),
                                   mask=lane0)
                return tuple(
                    jnp.where((ds_[v] == m) & (cis[v] == imin), BIG, ds_[v])
                    for v in range(len(ds_)))
            lax.fori_loop(0, KP1, sbody, cds)

            si0 = sib[pl.ds(0, 16)]
            si1 = sib[pl.ds(16, 16)]
            ivec = jnp.full((16,), i, i32)
            k16 = jnp.full((16,), 16, i32)
            for coord, buf in ((0, xb), (1, yb), (2, zb)):
                cvec = jnp.full((16,), coord, i32)
                v0 = plsc.load_gather(buf, [si0])
                v1 = plsc.load_gather(buf, [si1])
                plsc.store_scatter(ob, [lanes, ivec, cvec], v0)
                plsc.store_scatter(ob, [k16, ivec, cvec], v1, mask=lane0)
            return blk_base

        for blk in range(bpw):
            mloc = blk0 + blk
            lax.fori_loop(0, R, per_query, mloc * R)
            pltpu.sync_copy(ob, out_hbm.at[b * nb + mloc])

    return sc_kernel(ptsx, ptsy, ptsz)


def _bn_scale_shift(s, ss, cnt, g, b, eps=1e-5):
    mean = s / cnt
    var = ss / cnt - mean * mean
    scale = g / jnp.sqrt(var + eps)
    shift = b - mean * scale
    return scale.reshape(1, -1), shift.reshape(1, -1)


def _acc_init(i0_ref, i1_ref):
    @pl.when(pl.program_id(0) == 0)
    def _():
        i0_ref[...] = jnp.zeros_like(i0_ref)
        i1_ref[...] = jnp.zeros_like(i1_ref)


def _feat_y1_kernel(c_ref, w_ref, y1_ref, s_ref, ss_ref):
    c = c_ref[0][:, :, :3]
    ab = c[:1]
    rel = c[1:] - ab
    d = jnp.sqrt(jnp.sum(rel * rel, axis=2, keepdims=True) + 1e-8)
    feat = jnp.concatenate(
        (jnp.broadcast_to(ab, rel.shape), rel, d,
         jnp.zeros_like(d)), axis=2)
    y1 = jnp.dot(feat.reshape(16 * R, 8), w_ref[...],
                 preferred_element_type=jnp.float32)
    y1_ref[0] = y1.reshape(16, R, 16)
    _acc_init(s_ref, ss_ref)
    s_ref[...] += jnp.sum(y1, axis=0, keepdims=True)
    ss_ref[...] += jnp.sum(y1 * y1, axis=0, keepdims=True)


def _y2_kernel(y1_ref, w_ref, sc_ref, sh_ref, y2_ref, s_ref, ss_ref):
    h1 = jnp.maximum(y1_ref[0].reshape(16 * R, 16) * sc_ref[...]
                     + sh_ref[...], 0.0)
    y2 = jnp.dot(h1, w_ref[...], preferred_element_type=jnp.float32)
    y2_ref[0] = y2.reshape(16, R, 32)
    _acc_init(s_ref, ss_ref)
    s_ref[...] += jnp.sum(y2, axis=0, keepdims=True)
    ss_ref[...] += jnp.sum(y2 * y2, axis=0, keepdims=True)


def _pool_lift_kernel(y2_ref, pts_ref, w2_ref, sc_ref, sh_ref,
                      pool_ref, y3_ref, s_ref, ss_ref):
    h2 = jnp.maximum(y2_ref[0] * sc_ref[...].reshape(1, 1, 32)
                     + sh_ref[...].reshape(1, 1, 32), 0.0)
    pool_ref[0] = jnp.max(h2, axis=0)
    y3 = jnp.dot(pts_ref[0], w2_ref[...],
                 preferred_element_type=jnp.float32)
    y3_ref[0] = y3
    _acc_init(s_ref, ss_ref)
    s_ref[...] += jnp.sum(y3, axis=0, keepdims=True)
    ss_ref[...] += jnp.sum(y3 * y3, axis=0, keepdims=True)


def _y4_kernel(y3_ref, pool_ref, w3_ref, sc_ref, sh_ref,
               y4_ref, s_ref, ss_ref):
    lifted = jnp.maximum(y3_ref[0] * sc_ref[...] + sh_ref[...], 0.0)
    xcat = jnp.concatenate((lifted, pool_ref[0]), axis=1)
    y4 = jnp.dot(xcat, w3_ref[...], preferred_element_type=jnp.float32)
    y4_ref[0] = y4
    _acc_init(s_ref, ss_ref)
    s_ref[...] += jnp.sum(y4, axis=0, keepdims=True)
    ss_ref[...] += jnp.sum(y4 * y4, axis=0, keepdims=True)


def _final_kernel(y4_ref, sc_ref, sh_ref, o_ref):
    o_ref[0] = jnp.maximum(y4_ref[0] * sc_ref[...] + sh_ref[...], 0.0)


def _full_spec(*shape):
    nd = len(shape)
    return pl.BlockSpec(shape, lambda m: (0,) * nd)


def _mblk_spec(*rest):
    return pl.BlockSpec((1,) + rest, lambda m: (m,) + (0,) * len(rest))


def kernel(pts, W1a, g1a, b1a, W1b, g1b, b1b, W2, g2, b2, W3, g3, b3):
    Bb, Nn, _ = pts.shape
    nb = Nn // R
    M = Bb * nb
    cnt_nbr = float(Bb * Nn * 16)
    cnt_pts = float(Bb * Nn)
    coords = _sc_knn_coords(pts)
    pts_pad = jnp.pad(pts, ((0, 0), (0, 0), (0, 5))).reshape(M, R, 8)
    W1a_p = jnp.pad(W1a, ((0, 1), (0, 0)))
    W2_p = jnp.pad(W2, ((0, 5), (0, 0)))

    f32 = jnp.float32
    y1, s1, ss1 = pl.pallas_call(
        _feat_y1_kernel, grid=(M,),
        in_specs=[_mblk_spec(KP1, R, 4), _full_spec(8, 16)],
        out_specs=[_mblk_spec(16, R, 16), _full_spec(1, 16), _full_spec(1, 16)],
        out_shape=[jax.ShapeDtypeStruct((M, 16, R, 16), f32),
                   jax.ShapeDtypeStruct((1, 16), f32),
                   jax.ShapeDtypeStruct((1, 16), f32)],
    )(coords, W1a_p)
    sc1, sh1 = _bn_scale_shift(s1[0], ss1[0], cnt_nbr, g1a, b1a)

    y2, s2, ss2 = pl.pallas_call(
        _y2_kernel, grid=(M,),
        in_specs=[_mblk_spec(16, R, 16), _full_spec(16, 32),
                  _full_spec(1, 16), _full_spec(1, 16)],
        out_specs=[_mblk_spec(16, R, 32), _full_spec(1, 32), _full_spec(1, 32)],
        out_shape=[jax.ShapeDtypeStruct((M, 16, R, 32), f32),
                   jax.ShapeDtypeStruct((1, 32), f32),
                   jax.ShapeDtypeStruct((1, 32), f32)],
    )(y1, W1b, sc1, sh1)
    sc2, sh2 = _bn_scale_shift(s2[0], ss2[0], cnt_nbr, g1b, b1b)

    pooled, y3, s3, ss3 = pl.pallas_call(
        _pool_lift_kernel, grid=(M,),
        in_specs=[_mblk_spec(16, R, 32), _mblk_spec(R, 8),
                  _full_spec(8, 32), _full_spec(1, 32), _full_spec(1, 32)],
        out_specs=[_mblk_spec(R, 32), _mblk_spec(R, 32),
                   _full_spec(1, 32), _full_spec(1, 32)],
        out_shape=[jax.ShapeDtypeStruct((M, R, 32), f32),
                   jax.ShapeDtypeStruct((M, R, 32), f32),
                   jax.ShapeDtypeStruct((1, 32), f32),
                   jax.ShapeDtypeStruct((1, 32), f32)],
    )(y2, pts_pad, W2_p, sc2, sh2)
    sc3, sh3 = _bn_scale_shift(s3[0], ss3[0], cnt_pts, g2, b2)

    y4, s4, ss4 = pl.pallas_call(
        _y4_kernel, grid=(M,),
        in_specs=[_mblk_spec(R, 32), _mblk_spec(R, 32), _full_spec(64, 32),
                  _full_spec(1, 32), _full_spec(1, 32)],
        out_specs=[_mblk_spec(R, 32), _full_spec(1, 32), _full_spec(1, 32)],
        out_shape=[jax.ShapeDtypeStruct((M, R, 32), f32),
                   jax.ShapeDtypeStruct((1, 32), f32),
                   jax.ShapeDtypeStruct((1, 32), f32)],
    )(y3, pooled, W3, sc3, sh3)
    sc4, sh4 = _bn_scale_shift(s4[0], ss4[0], cnt_pts, g3, b3)

    out = pl.pallas_call(
        _final_kernel, grid=(M,),
        in_specs=[_mblk_spec(R, 32), _full_spec(1, 32), _full_spec(1, 32)],
        out_specs=_mblk_spec(R, 32),
        out_shape=jax.ShapeDtypeStruct((M, R, 32), f32),
    )(y4, sc4, sh4)
    return out.reshape(Bb, Nn, OUT_CHANNELS)


# speedup vs baseline: 1.9227x; 1.0815x over previous
import functools

import jax
import jax.numpy as jnp
from jax import lax
from jax.experimental import pallas as pl
from jax.experimental.pallas import tpu as pltpu
from jax.experimental.pallas import tpu_sc as plsc

NUM_NEIGHBORS = 16
OUT_CHANNELS = 32
KP1 = NUM_NEIGHBORS + 1
R = 256
BIG = 3.0e38


def _knn_kernel(rows_ref, ptsT_ref, ptsP_ref, out_ref, scratch_ref):
    rows = rows_ref[0]
    ptsT = ptsT_ref[0]
    ptsP = ptsP_ref[0]
    n = ptsT.shape[1]
    sqr = jnp.sum(rows * rows, axis=1, keepdims=True)
    sqc = jnp.sum(ptsT * ptsT, axis=0, keepdims=True)
    dot = jnp.dot(rows, ptsT, preferred_element_type=jnp.float32)
    dist = (sqr + sqc) - 2.0 * dot
    iota = jax.lax.broadcasted_iota(jnp.int32, (R, n), 1)

    def body(k, d):
        minv = jnp.min(d, axis=1, keepdims=True)
        hit = d == minv
        idxv = jnp.min(jnp.where(hit, iota, n), axis=1, keepdims=True)
        sel = iota == idxv
        coords = jax.lax.dot_general(
            sel.astype(jnp.float32), ptsP,
            (((1,), (0,)), ((), ())),
            precision=jax.lax.Precision.HIGHEST,
            preferred_element_type=jnp.float32)
        scratch_ref[k] = coords
        return jnp.where(sel, BIG, d)

    jax.lax.fori_loop(0, KP1, body, dist, unroll=False)
    out_ref[0] = scratch_ref[...]


def _knn_coords(pts):
    Bb, Nn, _ = pts.shape
    pts_pad = jnp.pad(pts, ((0, 0), (0, 0), (0, 5)))
    ptsT = jnp.transpose(pts_pad, (0, 2, 1))
    nb = Nn // R
    return pl.pallas_call(
        _knn_kernel,
        grid=(Bb, nb),
        in_specs=[
            pl.BlockSpec((1, R, 8), lambda b, i: (b, i, 0)),
            pl.BlockSpec((1, 8, Nn), lambda b, i: (b, 0, 0)),
            pl.BlockSpec((1, Nn, 8), lambda b, i: (b, 0, 0)),
        ],
        out_specs=pl.BlockSpec((1, KP1, R, 8), lambda b, i: (b * nb + i, 0, 0, 0)),
        out_shape=jax.ShapeDtypeStruct((Bb * nb, KP1, R, 8), jnp.float32),
        scratch_shapes=[pltpu.VMEM((KP1, R, 8), jnp.float32)],
    )(pts_pad, ptsT, pts_pad)


CAND = 128
MAXI = 2147483647
NEG = -3.0e38


def _sc_knn_coords(pts):
    Bb, Nn, _ = pts.shape
    nb = Nn // R
    M = Bb * nb
    wpb = 32 // Bb
    bpw = nb // wpb
    ptsx = pts[:, :, 0]
    ptsy = pts[:, :, 1]
    ptsz = pts[:, :, 2]
    f32, i32 = jnp.float32, jnp.int32
    mesh = plsc.VectorSubcoreMesh(core_axis_name="c", subcore_axis_name="s")

    @functools.partial(
        pl.kernel,
        out_type=jax.ShapeDtypeStruct((M, KP1, R, 4), f32),
        mesh=mesh,
        compiler_params=pltpu.CompilerParams(
            needs_layout_passes=False, use_tc_tiling_on_sc=False),
        scratch_types=[
            pltpu.VMEM((Nn,), f32),
            pltpu.VMEM((Nn,), f32),
            pltpu.VMEM((Nn,), f32),
            pltpu.VMEM((Nn,), f32),
            pltpu.VMEM((Nn,), f32),
            pltpu.VMEM((Nn,), f32),
            pltpu.VMEM((Nn,), f32),
            pltpu.VMEM((Nn,), f32),
            pltpu.VMEM((32,), f32),
            pltpu.VMEM((CAND + 16,), f32),
            pltpu.VMEM((CAND + 16,), i32),
            pltpu.VMEM((48,), i32),
            pltpu.VMEM((KP1, R, 4), f32),
        ],
    )
    def sc_kernel(x_hbm, y_hbm, z_hbm, out_hbm,
                  xb, yb, zb, xr, yr, zr, sqb, db, gmb, cdb, cib, sib, ob):
        cid = lax.axis_index("c")
        sid = lax.axis_index("s")
        w = sid * 2 + cid
        b = w // wpb
        blk0 = (w % wpb) * bpw
        pltpu.sync_copy(x_hbm.at[b], xb)
        pltpu.sync_copy(y_hbm.at[b], yb)
        pltpu.sync_copy(z_hbm.at[b], zb)
        lanes = lax.iota(i32, 16)
        lane0 = lanes == 0
        big = jnp.full((16,), BIG, f32)

        def _rnd_bf16(x):
            u = plsc.bitcast(x, i32)
            u = (u + 0x7FFF + ((u >> 16) & 1)) & ~0xFFFF
            return plsc.bitcast(u, f32)

        def prep(j, _):
            sl = pl.ds(j * 16, 16)
            x = xb[sl]
            y = yb[sl]
            z = zb[sl]
            xr[sl] = _rnd_bf16(x)
            yr[sl] = _rnd_bf16(y)
            zr[sl] = _rnd_bf16(z)
            sqb[sl] = (x * x + y * y) + z * z
            return 0
        lax.fori_loop(0, Nn // 16, prep, 0)

        def per_query(i, blk_base):
            qi = blk_base + i
            qiv = jnp.full((16,), qi, i32)
            qx2 = plsc.load_gather(xr, [qiv]) * 2.0
            qy2 = plsc.load_gather(yr, [qiv]) * 2.0
            qz2 = plsc.load_gather(zr, [qiv]) * 2.0

            def gbody(g, _):
                def inner(jj, acc):
                    off = g * 128 + jj * 16
                    sl = pl.ds(off, 16)
                    dot2 = (qx2 * xr[sl] + qy2 * yr[sl]) + qz2 * zr[sl]
                    d = sqb[sl] - dot2
                    db[sl] = d
                    return jnp.minimum(acc, d)
                acc = lax.fori_loop(0, 8, inner, big, unroll=True)
                plsc.store_scatter(gmb, [jnp.full((16,), g, i32)],
                                   jnp.full((16,), jnp.min(acc), f32),
                                   mask=lane0)
                return 0
            lax.fori_loop(0, 32, gbody, 0)

            def tbody(k, carry):
                g0, g1, _ = carry
                m = jnp.min(jnp.minimum(g0, g1))
                return (jnp.where(g0 == m, BIG, g0),
                        jnp.where(g1 == m, BIG, g1), m)
            _, _, T = lax.fori_loop(
                0, KP1, tbody,
                (gmb[pl.ds(0, 16)], gmb[pl.ds(16, 16)], 0.0))

            negv = jnp.full((16,), NEG, f32)
            for v in range(CAND // 16 + 1):
                cdb[pl.ds(v * 16, 16)] = negv

            def cgroup(g, off):
                def do(off):
                    def inner(jj, off):
                        j = g * 8 + jj
                        d = db[pl.ds(j * 16, 16)]
                        msk = d <= T
                        plsc.store_compressed(
                            cdb.at[pl.ds(off, 16)], d, mask=msk)
                        plsc.store_compressed(
                            cib.at[pl.ds(off, 16)], lanes + j * 16, mask=msk)
                        cnt = jnp.max(plsc.all_reduce_population_count(msk))
                        return jnp.minimum(off + cnt, CAND)
                    return lax.fori_loop(0, 8, inner, off)
                gm = jnp.min(plsc.load_gather(gmb, [jnp.full((16,), g, i32)]))
                return lax.cond(gm <= T, do, lambda o: o, off)
            cnt = lax.fori_loop(0, 32, cgroup, 0)
            nv = (cnt + 15) // 16

            def cull(_, c):
                def vmax(v, mm):
                    return jnp.maximum(mm, jnp.max(cdb[pl.ds(v * 16, 16)]))
                m = lax.fori_loop(0, nv, vmax, NEG)

                def vidx(v, ii):
                    d = cdb[pl.ds(v * 16, 16)]
                    return jnp.maximum(ii, jnp.max(
                        jnp.where(d == m, cib[pl.ds(v * 16, 16)], -1)))
                imax = lax.fori_loop(0, nv, vidx, -1)

                def vclr(v, _):
                    sl = pl.ds(v * 16, 16)
                    d = cdb[sl]
                    hit = (d == m) & (cib[sl] == imax)
                    cdb[sl] = jnp.where(hit, NEG, d)
                    return 0
                lax.fori_loop(0, nv, vclr, 0)
                return 0
            lax.fori_loop(0, cnt - KP1, cull, 0)

            def vmin(v, mm):
                d = cdb[pl.ds(v * 16, 16)]
                return jnp.minimum(mm, jnp.min(jnp.where(d > NEG, d, BIG)))
            m0 = lax.fori_loop(0, nv, vmin, BIG)

            def vi0(v, ii):
                d = cdb[pl.ds(v * 16, 16)]
                return jnp.minimum(ii, jnp.min(
                    jnp.where(d == m0, cib[pl.ds(v * 16, 16)], MAXI)))
            i0 = lax.fori_loop(0, nv, vi0, MAXI)

            def coll(v, off2):
                sl = pl.ds(v * 16, 16)
                d = cdb[sl]
                ci = cib[sl]
                msk = (d > NEG) & ((d != m0) | (ci != i0))
                plsc.store_compressed(sib.at[pl.ds(off2, 16)], ci, mask=msk)
                return off2 + jnp.max(plsc.all_reduce_population_count(msk))
            lax.fori_loop(0, nv, coll, 0)

            si = sib[pl.ds(0, 16)]
            i0v = jnp.full((16,), i0, i32)
            ivec = jnp.full((16,), i, i32)
            zv = jnp.zeros((16,), i32)
            for coord, buf in ((0, xb), (1, yb), (2, zb)):
                cvec = jnp.full((16,), coord, i32)
                v1 = plsc.load_gather(buf, [si])
                v0 = plsc.load_gather(buf, [i0v])
                plsc.store_scatter(ob, [lanes + 1, ivec, cvec], v1)
                plsc.store_scatter(ob, [zv, ivec, cvec], v0, mask=lane0)
            return blk_base

        for blk in range(bpw):
            mloc = blk0 + blk
            lax.fori_loop(0, R, per_query, mloc * R)
            pltpu.sync_copy(ob, out_hbm.at[b * nb + mloc])

    return sc_kernel(ptsx, ptsy, ptsz)


def _bn_scale_shift(s, ss, cnt, g, b, eps=1e-5):
    mean = s / cnt
    var = ss / cnt - mean * mean
    scale = g / jnp.sqrt(var + eps)
    shift = b - mean * scale
    return scale.reshape(1, -1), shift.reshape(1, -1)


def _acc_init(i0_ref, i1_ref):
    @pl.when(pl.program_id(0) == 0)
    def _():
        i0_ref[...] = jnp.zeros_like(i0_ref)
        i1_ref[...] = jnp.zeros_like(i1_ref)


def _feat_y1_kernel(c_ref, w_ref, y1_ref, s_ref, ss_ref):
    c = c_ref[0][:, :, :3]
    ab = c[:1]
    rel = c[1:] - ab
    d = jnp.sqrt(jnp.sum(rel * rel, axis=2, keepdims=True) + 1e-8)
    feat = jnp.concatenate(
        (jnp.broadcast_to(ab, rel.shape), rel, d,
         jnp.zeros_like(d)), axis=2)
    y1 = jnp.dot(feat.reshape(16 * R, 8), w_ref[...],
                 preferred_element_type=jnp.float32)
    y1_ref[0] = y1.reshape(16, R, 16)
    _acc_init(s_ref, ss_ref)
    s_ref[...] += jnp.sum(y1, axis=0, keepdims=True)
    ss_ref[...] += jnp.sum(y1 * y1, axis=0, keepdims=True)


def _y2_kernel(y1_ref, w_ref, sc_ref, sh_ref, y2_ref, s_ref, ss_ref):
    h1 = jnp.maximum(y1_ref[0].reshape(16 * R, 16) * sc_ref[...]
                     + sh_ref[...], 0.0)
    y2 = jnp.dot(h1, w_ref[...], preferred_element_type=jnp.float32)
    y2_ref[0] = y2.reshape(16, R, 32)
    _acc_init(s_ref, ss_ref)
    s_ref[...] += jnp.sum(y2, axis=0, keepdims=True)
    ss_ref[...] += jnp.sum(y2 * y2, axis=0, keepdims=True)


def _pool_lift_kernel(y2_ref, pts_ref, w2_ref, sc_ref, sh_ref,
                      pool_ref, y3_ref, s_ref, ss_ref):
    h2 = jnp.maximum(y2_ref[0] * sc_ref[...].reshape(1, 1, 32)
                     + sh_ref[...].reshape(1, 1, 32), 0.0)
    pool_ref[0] = jnp.max(h2, axis=0)
    y3 = jnp.dot(pts_ref[0], w2_ref[...],
                 preferred_element_type=jnp.float32)
    y3_ref[0] = y3
    _acc_init(s_ref, ss_ref)
    s_ref[...] += jnp.sum(y3, axis=0, keepdims=True)
    ss_ref[...] += jnp.sum(y3 * y3, axis=0, keepdims=True)


def _y4_kernel(y3_ref, pool_ref, w3_ref, sc_ref, sh_ref,
               y4_ref, s_ref, ss_ref):
    lifted = jnp.maximum(y3_ref[0] * sc_ref[...] + sh_ref[...], 0.0)
    xcat = jnp.concatenate((lifted, pool_ref[0]), axis=1)
    y4 = jnp.dot(xcat, w3_ref[...], preferred_element_type=jnp.float32)
    y4_ref[0] = y4
    _acc_init(s_ref, ss_ref)
    s_ref[...] += jnp.sum(y4, axis=0, keepdims=True)
    ss_ref[...] += jnp.sum(y4 * y4, axis=0, keepdims=True)


def _final_kernel(y4_ref, sc_ref, sh_ref, o_ref):
    o_ref[0] = jnp.maximum(y4_ref[0] * sc_ref[...] + sh_ref[...], 0.0)


def _full_spec(*shape):
    nd = len(shape)
    return pl.BlockSpec(shape, lambda m: (0,) * nd)


def _mblk_spec(*rest):
    return pl.BlockSpec((1,) + rest, lambda m: (m,) + (0,) * len(rest))


def kernel(pts, W1a, g1a, b1a, W1b, g1b, b1b, W2, g2, b2, W3, g3, b3):
    Bb, Nn, _ = pts.shape
    nb = Nn // R
    M = Bb * nb
    cnt_nbr = float(Bb * Nn * 16)
    cnt_pts = float(Bb * Nn)
    coords = _sc_knn_coords(pts)
    pts_pad = jnp.pad(pts, ((0, 0), (0, 0), (0, 5))).reshape(M, R, 8)
    W1a_p = jnp.pad(W1a, ((0, 1), (0, 0)))
    W2_p = jnp.pad(W2, ((0, 5), (0, 0)))

    f32 = jnp.float32
    y1, s1, ss1 = pl.pallas_call(
        _feat_y1_kernel, grid=(M,),
        in_specs=[_mblk_spec(KP1, R, 4), _full_spec(8, 16)],
        out_specs=[_mblk_spec(16, R, 16), _full_spec(1, 16), _full_spec(1, 16)],
        out_shape=[jax.ShapeDtypeStruct((M, 16, R, 16), f32),
                   jax.ShapeDtypeStruct((1, 16), f32),
                   jax.ShapeDtypeStruct((1, 16), f32)],
    )(coords, W1a_p)
    sc1, sh1 = _bn_scale_shift(s1[0], ss1[0], cnt_nbr, g1a, b1a)

    y2, s2, ss2 = pl.pallas_call(
        _y2_kernel, grid=(M,),
        in_specs=[_mblk_spec(16, R, 16), _full_spec(16, 32),
                  _full_spec(1, 16), _full_spec(1, 16)],
        out_specs=[_mblk_spec(16, R, 32), _full_spec(1, 32), _full_spec(1, 32)],
        out_shape=[jax.ShapeDtypeStruct((M, 16, R, 32), f32),
                   jax.ShapeDtypeStruct((1, 32), f32),
                   jax.ShapeDtypeStruct((1, 32), f32)],
    )(y1, W1b, sc1, sh1)
    sc2, sh2 = _bn_scale_shift(s2[0], ss2[0], cnt_nbr, g1b, b1b)

    pooled, y3, s3, ss3 = pl.pallas_call(
        _pool_lift_kernel, grid=(M,),
        in_specs=[_mblk_spec(16, R, 32), _mblk_spec(R, 8),
                  _full_spec(8, 32), _full_spec(1, 32), _full_spec(1, 32)],
        out_specs=[_mblk_spec(R, 32), _mblk_spec(R, 32),
                   _full_spec(1, 32), _full_spec(1, 32)],
        out_shape=[jax.ShapeDtypeStruct((M, R, 32), f32),
                   jax.ShapeDtypeStruct((M, R, 32), f32),
                   jax.ShapeDtypeStruct((1, 32), f32),
                   jax.ShapeDtypeStruct((1, 32), f32)],
    )(y2, pts_pad, W2_p, sc2, sh2)
    sc3, sh3 = _bn_scale_shift(s3[0], ss3[0], cnt_pts, g2, b2)

    y4, s4, ss4 = pl.pallas_call(
        _y4_kernel, grid=(M,),
        in_specs=[_mblk_spec(R, 32), _mblk_spec(R, 32), _full_spec(64, 32),
                  _full_spec(1, 32), _full_spec(1, 32)],
        out_specs=[_mblk_spec(R, 32), _full_spec(1, 32), _full_spec(1, 32)],
        out_shape=[jax.ShapeDtypeStruct((M, R, 32), f32),
                   jax.ShapeDtypeStruct((1, 32), f32),
                   jax.ShapeDtypeStruct((1, 32), f32)],
    )(y3, pooled, W3, sc3, sh3)
    sc4, sh4 = _bn_scale_shift(s4[0], ss4[0], cnt_pts, g3, b3)

    out = pl.pallas_call(
        _final_kernel, grid=(M,),
        in_specs=[_mblk_spec(R, 32), _full_spec(1, 32), _full_spec(1, 32)],
        out_specs=_mblk_spec(R, 32),
        out_shape=jax.ShapeDtypeStruct((M, R, 32), f32),
    )(y4, sc4, sh4)
    return out.reshape(Bb, Nn, OUT_CHANNELS)


# speedup vs baseline: 2.4677x; 1.2835x over previous
import functools

import jax
import jax.numpy as jnp
from jax import lax
from jax.experimental import pallas as pl
from jax.experimental.pallas import tpu as pltpu
from jax.experimental.pallas import tpu_sc as plsc

NUM_NEIGHBORS = 16
OUT_CHANNELS = 32
KP1 = NUM_NEIGHBORS + 1
R = 256
BIG = 3.0e38


def _knn_kernel(rows_ref, ptsT_ref, ptsP_ref, out_ref, scratch_ref):
    rows = rows_ref[0]
    ptsT = ptsT_ref[0]
    ptsP = ptsP_ref[0]
    n = ptsT.shape[1]
    sqr = jnp.sum(rows * rows, axis=1, keepdims=True)
    sqc = jnp.sum(ptsT * ptsT, axis=0, keepdims=True)
    dot = jnp.dot(rows, ptsT, preferred_element_type=jnp.float32)
    dist = (sqr + sqc) - 2.0 * dot
    iota = jax.lax.broadcasted_iota(jnp.int32, (R, n), 1)

    def body(k, d):
        minv = jnp.min(d, axis=1, keepdims=True)
        hit = d == minv
        idxv = jnp.min(jnp.where(hit, iota, n), axis=1, keepdims=True)
        sel = iota == idxv
        coords = jax.lax.dot_general(
            sel.astype(jnp.float32), ptsP,
            (((1,), (0,)), ((), ())),
            precision=jax.lax.Precision.HIGHEST,
            preferred_element_type=jnp.float32)
        scratch_ref[k] = coords
        return jnp.where(sel, BIG, d)

    jax.lax.fori_loop(0, KP1, body, dist, unroll=False)
    out_ref[0] = scratch_ref[...]


def _knn_coords(pts):
    Bb, Nn, _ = pts.shape
    pts_pad = jnp.pad(pts, ((0, 0), (0, 0), (0, 5)))
    ptsT = jnp.transpose(pts_pad, (0, 2, 1))
    nb = Nn // R
    return pl.pallas_call(
        _knn_kernel,
        grid=(Bb, nb),
        in_specs=[
            pl.BlockSpec((1, R, 8), lambda b, i: (b, i, 0)),
            pl.BlockSpec((1, 8, Nn), lambda b, i: (b, 0, 0)),
            pl.BlockSpec((1, Nn, 8), lambda b, i: (b, 0, 0)),
        ],
        out_specs=pl.BlockSpec((1, KP1, R, 8), lambda b, i: (b * nb + i, 0, 0, 0)),
        out_shape=jax.ShapeDtypeStruct((Bb * nb, KP1, R, 8), jnp.float32),
        scratch_shapes=[pltpu.VMEM((KP1, R, 8), jnp.float32)],
    )(pts_pad, ptsT, pts_pad)


CAND = 128
MAXI = 2147483647
NEG = -3.0e38


def _sc_knn_coords(pts):
    Bb, Nn, _ = pts.shape
    nb = Nn // R
    M = Bb * nb
    wpb = 32 // Bb
    bpw = nb // wpb
    ptsx = pts[:, :, 0]
    ptsy = pts[:, :, 1]
    ptsz = pts[:, :, 2]
    f32, i32 = jnp.float32, jnp.int32
    mesh = plsc.VectorSubcoreMesh(core_axis_name="c", subcore_axis_name="s")

    @functools.partial(
        pl.kernel,
        out_type=jax.ShapeDtypeStruct((M, KP1, R, 4), f32),
        mesh=mesh,
        compiler_params=pltpu.CompilerParams(
            needs_layout_passes=False, use_tc_tiling_on_sc=False),
        scratch_types=[
            pltpu.VMEM((Nn,), f32),
            pltpu.VMEM((Nn,), f32),
            pltpu.VMEM((Nn,), f32),
            pltpu.VMEM((Nn,), f32),
            pltpu.VMEM((Nn,), f32),
            pltpu.VMEM((Nn,), f32),
            pltpu.VMEM((Nn,), f32),
            pltpu.VMEM((Nn,), f32),
            pltpu.VMEM((32,), f32),
            pltpu.VMEM((CAND + 16,), f32),
            pltpu.VMEM((CAND + 16,), i32),
            pltpu.VMEM((48,), i32),
            pltpu.VMEM((KP1, R, 4), f32),
        ],
    )
    def sc_kernel(x_hbm, y_hbm, z_hbm, out_hbm,
                  xb, yb, zb, xr, yr, zr, sqb, db, gmb, cdb, cib, sib, ob):
        cid = lax.axis_index("c")
        sid = lax.axis_index("s")
        w = sid * 2 + cid
        b = w // wpb
        blk0 = (w % wpb) * bpw
        pltpu.sync_copy(x_hbm.at[b], xb)
        pltpu.sync_copy(y_hbm.at[b], yb)
        pltpu.sync_copy(z_hbm.at[b], zb)
        lanes = lax.iota(i32, 16)
        lane0 = lanes == 0
        big = jnp.full((16,), BIG, f32)

        def _rnd_bf16(x):
            u = plsc.bitcast(x, i32)
            u = (u + 0x7FFF + ((u >> 16) & 1)) & ~0xFFFF
            return plsc.bitcast(u, f32)

        def prep(j, _):
            sl = pl.ds(j * 16, 16)
            x = xb[sl]
            y = yb[sl]
            z = zb[sl]
            xr[sl] = _rnd_bf16(x)
            yr[sl] = _rnd_bf16(y)
            zr[sl] = _rnd_bf16(z)
            sqb[sl] = (x * x + y * y) + z * z
            return 0
        lax.fori_loop(0, Nn // 16, prep, 0)

        def per_query(i, blk_base):
            qi = blk_base + i
            qiv = jnp.full((16,), qi, i32)
            qx2 = plsc.load_gather(xr, [qiv]) * 2.0
            qy2 = plsc.load_gather(yr, [qiv]) * 2.0
            qz2 = plsc.load_gather(zr, [qiv]) * 2.0

            def gbody(g, _):
                def inner(jj, acc):
                    sl = pl.ds(jj, 16)
                    dot2 = (qx2 * xr[sl] + qy2 * yr[sl]) + qz2 * zr[sl]
                    d = sqb[sl] - dot2
                    db[sl] = d
                    return jnp.minimum(acc, d)
                acc = plsc.parallel_loop(
                    g * 128, g * 128 + 128, 16, unroll=8, carry=big)(inner)
                plsc.store_scatter(gmb, [jnp.full((16,), g, i32)],
                                   jnp.full((16,), jnp.min(acc), f32),
                                   mask=lane0)
                return 0
            lax.fori_loop(0, 32, gbody, 0)

            def tbody(k, carry):
                g0, g1, _ = carry
                m = jnp.min(jnp.minimum(g0, g1))
                return (jnp.where(g0 == m, BIG, g0),
                        jnp.where(g1 == m, BIG, g1), m)
            _, _, T = lax.fori_loop(
                0, KP1, tbody,
                (gmb[pl.ds(0, 16)], gmb[pl.ds(16, 16)], 0.0))

            negv = jnp.full((16,), NEG, f32)
            for v in range(CAND // 16 + 1):
                cdb[pl.ds(v * 16, 16)] = negv

            def cgroup(g, off):
                def do(off):
                    def inner(jj, off):
                        j = g * 8 + jj
                        d = db[pl.ds(j * 16, 16)]
                        msk = d <= T
                        plsc.store_compressed(
                            cdb.at[pl.ds(off, 16)], d, mask=msk)
                        plsc.store_compressed(
                            cib.at[pl.ds(off, 16)], lanes + j * 16, mask=msk)
                        cnt = jnp.max(plsc.all_reduce_population_count(msk))
                        return jnp.minimum(off + cnt, CAND)
                    return lax.fori_loop(0, 8, inner, off)
                gm = jnp.min(plsc.load_gather(gmb, [jnp.full((16,), g, i32)]))
                return lax.cond(gm <= T, do, lambda o: o, off)
            cnt = lax.fori_loop(0, 32, cgroup, 0)
            nv = (cnt + 15) // 16

            def cull(_, c):
                def vmax(v, mm):
                    return jnp.maximum(mm, jnp.max(cdb[pl.ds(v * 16, 16)]))
                m = lax.fori_loop(0, nv, vmax, NEG)

                def vidx(v, ii):
                    d = cdb[pl.ds(v * 16, 16)]
                    return jnp.maximum(ii, jnp.max(
                        jnp.where(d == m, cib[pl.ds(v * 16, 16)], -1)))
                imax = lax.fori_loop(0, nv, vidx, -1)

                def vclr(v, _):
                    sl = pl.ds(v * 16, 16)
                    d = cdb[sl]
                    hit = (d == m) & (cib[sl] == imax)
                    cdb[sl] = jnp.where(hit, NEG, d)
                    return 0
                lax.fori_loop(0, nv, vclr, 0)
                return 0
            lax.fori_loop(0, cnt - KP1, cull, 0)

            def vmin(v, mm):
                d = cdb[pl.ds(v * 16, 16)]
                return jnp.minimum(mm, jnp.min(jnp.where(d > NEG, d, BIG)))
            m0 = lax.fori_loop(0, nv, vmin, BIG)

            def vi0(v, ii):
                d = cdb[pl.ds(v * 16, 16)]
                return jnp.minimum(ii, jnp.min(
                    jnp.where(d == m0, cib[pl.ds(v * 16, 16)], MAXI)))
            i0 = lax.fori_loop(0, nv, vi0, MAXI)

            def coll(v, off2):
                sl = pl.ds(v * 16, 16)
                d = cdb[sl]
                ci = cib[sl]
                msk = (d > NEG) & ((d != m0) | (ci != i0))
                plsc.store_compressed(sib.at[pl.ds(off2, 16)], ci, mask=msk)
                return off2 + jnp.max(plsc.all_reduce_population_count(msk))
            lax.fori_loop(0, nv, coll, 0)

            si = sib[pl.ds(0, 16)]
            i0v = jnp.full((16,), i0, i32)
            ivec = jnp.full((16,), i, i32)
            zv = jnp.zeros((16,), i32)
            for coord, buf in ((0, xb), (1, yb), (2, zb)):
                cvec = jnp.full((16,), coord, i32)
                v1 = plsc.load_gather(buf, [si])
                v0 = plsc.load_gather(buf, [i0v])
                plsc.store_scatter(ob, [lanes + 1, ivec, cvec], v1)
                plsc.store_scatter(ob, [zv, ivec, cvec], v0, mask=lane0)
            return blk_base

        for blk in range(bpw):
            mloc = blk0 + blk
            lax.fori_loop(0, R, per_query, mloc * R)
            pltpu.sync_copy(ob, out_hbm.at[b * nb + mloc])

    return sc_kernel(ptsx, ptsy, ptsz)


def _bn_scale_shift(s, ss, cnt, g, b, eps=1e-5):
    mean = s / cnt
    var = ss / cnt - mean * mean
    scale = g / jnp.sqrt(var + eps)
    shift = b - mean * scale
    return scale.reshape(1, -1), shift.reshape(1, -1)


def _acc_init(i0_ref, i1_ref):
    @pl.when(pl.program_id(0) == 0)
    def _():
        i0_ref[...] = jnp.zeros_like(i0_ref)
        i1_ref[...] = jnp.zeros_like(i1_ref)


def _feat_y1_kernel(c_ref, w_ref, y1_ref, s_ref, ss_ref):
    c = c_ref[0][:, :, :3]
    ab = c[:1]
    rel = c[1:] - ab
    d = jnp.sqrt(jnp.sum(rel * rel, axis=2, keepdims=True) + 1e-8)
    feat = jnp.concatenate(
        (jnp.broadcast_to(ab, rel.shape), rel, d,
         jnp.zeros_like(d)), axis=2)
    y1 = jnp.dot(feat.reshape(16 * R, 8), w_ref[...],
                 preferred_element_type=jnp.float32)
    y1_ref[0] = y1.reshape(16, R, 16)
    _acc_init(s_ref, ss_ref)
    s_ref[...] += jnp.sum(y1, axis=0, keepdims=True)
    ss_ref[...] += jnp.sum(y1 * y1, axis=0, keepdims=True)


def _y2_kernel(y1_ref, w_ref, sc_ref, sh_ref, y2_ref, s_ref, ss_ref):
    h1 = jnp.maximum(y1_ref[0].reshape(16 * R, 16) * sc_ref[...]
                     + sh_ref[...], 0.0)
    y2 = jnp.dot(h1, w_ref[...], preferred_element_type=jnp.float32)
    y2_ref[0] = y2.reshape(16, R, 32)
    _acc_init(s_ref, ss_ref)
    s_ref[...] += jnp.sum(y2, axis=0, keepdims=True)
    ss_ref[...] += jnp.sum(y2 * y2, axis=0, keepdims=True)


def _pool_lift_kernel(y2_ref, pts_ref, w2_ref, sc_ref, sh_ref,
                      pool_ref, y3_ref, s_ref, ss_ref):
    h2 = jnp.maximum(y2_ref[0] * sc_ref[...].reshape(1, 1, 32)
                     + sh_ref[...].reshape(1, 1, 32), 0.0)
    pool_ref[0] = jnp.max(h2, axis=0)
    y3 = jnp.dot(pts_ref[0], w2_ref[...],
                 preferred_element_type=jnp.float32)
    y3_ref[0] = y3
    _acc_init(s_ref, ss_ref)
    s_ref[...] += jnp.sum(y3, axis=0, keepdims=True)
    ss_ref[...] += jnp.sum(y3 * y3, axis=0, keepdims=True)


def _y4_kernel(y3_ref, pool_ref, w3_ref, sc_ref, sh_ref,
               y4_ref, s_ref, ss_ref):
    lifted = jnp.maximum(y3_ref[0] * sc_ref[...] + sh_ref[...], 0.0)
    xcat = jnp.concatenate((lifted, pool_ref[0]), axis=1)
    y4 = jnp.dot(xcat, w3_ref[...], preferred_element_type=jnp.float32)
    y4_ref[0] = y4
    _acc_init(s_ref, ss_ref)
    s_ref[...] += jnp.sum(y4, axis=0, keepdims=True)
    ss_ref[...] += jnp.sum(y4 * y4, axis=0, keepdims=True)


def _final_kernel(y4_ref, sc_ref, sh_ref, o_ref):
    o_ref[0] = jnp.maximum(y4_ref[0] * sc_ref[...] + sh_ref[...], 0.0)


def _full_spec(*shape):
    nd = len(shape)
    return pl.BlockSpec(shape, lambda m: (0,) * nd)


def _mblk_spec(*rest):
    return pl.BlockSpec((1,) + rest, lambda m: (m,) + (0,) * len(rest))


def kernel(pts, W1a, g1a, b1a, W1b, g1b, b1b, W2, g2, b2, W3, g3, b3):
    Bb, Nn, _ = pts.shape
    nb = Nn // R
    M = Bb * nb
    cnt_nbr = float(Bb * Nn * 16)
    cnt_pts = float(Bb * Nn)
    coords = _sc_knn_coords(pts)
    pts_pad = jnp.pad(pts, ((0, 0), (0, 0), (0, 5))).reshape(M, R, 8)
    W1a_p = jnp.pad(W1a, ((0, 1), (0, 0)))
    W2_p = jnp.pad(W2, ((0, 5), (0, 0)))

    f32 = jnp.float32
    y1, s1, ss1 = pl.pallas_call(
        _feat_y1_kernel, grid=(M,),
        in_specs=[_mblk_spec(KP1, R, 4), _full_spec(8, 16)],
        out_specs=[_mblk_spec(16, R, 16), _full_spec(1, 16), _full_spec(1, 16)],
        out_shape=[jax.ShapeDtypeStruct((M, 16, R, 16), f32),
                   jax.ShapeDtypeStruct((1, 16), f32),
                   jax.ShapeDtypeStruct((1, 16), f32)],
    )(coords, W1a_p)
    sc1, sh1 = _bn_scale_shift(s1[0], ss1[0], cnt_nbr, g1a, b1a)

    y2, s2, ss2 = pl.pallas_call(
        _y2_kernel, grid=(M,),
        in_specs=[_mblk_spec(16, R, 16), _full_spec(16, 32),
                  _full_spec(1, 16), _full_spec(1, 16)],
        out_specs=[_mblk_spec(16, R, 32), _full_spec(1, 32), _full_spec(1, 32)],
        out_shape=[jax.ShapeDtypeStruct((M, 16, R, 32), f32),
                   jax.ShapeDtypeStruct((1, 32), f32),
                   jax.ShapeDtypeStruct((1, 32), f32)],
    )(y1, W1b, sc1, sh1)
    sc2, sh2 = _bn_scale_shift(s2[0], ss2[0], cnt_nbr, g1b, b1b)

    pooled, y3, s3, ss3 = pl.pallas_call(
        _pool_lift_kernel, grid=(M,),
        in_specs=[_mblk_spec(16, R, 32), _mblk_spec(R, 8),
                  _full_spec(8, 32), _full_spec(1, 32), _full_spec(1, 32)],
        out_specs=[_mblk_spec(R, 32), _mblk_spec(R, 32),
                   _full_spec(1, 32), _full_spec(1, 32)],
        out_shape=[jax.ShapeDtypeStruct((M, R, 32), f32),
                   jax.ShapeDtypeStruct((M, R, 32), f32),
                   jax.ShapeDtypeStruct((1, 32), f32),
                   jax.ShapeDtypeStruct((1, 32), f32)],
    )(y2, pts_pad, W2_p, sc2, sh2)
    sc3, sh3 = _bn_scale_shift(s3[0], ss3[0], cnt_pts, g2, b2)

    y4, s4, ss4 = pl.pallas_call(
        _y4_kernel, grid=(M,),
        in_specs=[_mblk_spec(R, 32), _mblk_spec(R, 32), _full_spec(64, 32),
                  _full_spec(1, 32), _full_spec(1, 32)],
        out_specs=[_mblk_spec(R, 32), _full_spec(1, 32), _full_spec(1, 32)],
        out_shape=[jax.ShapeDtypeStruct((M, R, 32), f32),
                   jax.ShapeDtypeStruct((1, 32), f32),
                   jax.ShapeDtypeStruct((1, 32), f32)],
    )(y3, pooled, W3, sc3, sh3)
    sc4, sh4 = _bn_scale_shift(s4[0], ss4[0], cnt_pts, g3, b3)

    out = pl.pallas_call(
        _final_kernel, grid=(M,),
        in_specs=[_mblk_spec(R, 32), _full_spec(1, 32), _full_spec(1, 32)],
        out_specs=_mblk_spec(R, 32),
        out_shape=jax.ShapeDtypeStruct((M, R, 32), f32),
    )(y4, sc4, sh4)
    return out.reshape(Bb, Nn, OUT_CHANNELS)


# speedup vs baseline: 3.1084x; 1.2596x over previous
import functools

import jax
import jax.numpy as jnp
from jax import lax
from jax.experimental import pallas as pl
from jax.experimental.pallas import tpu as pltpu
from jax.experimental.pallas import tpu_sc as plsc

NUM_NEIGHBORS = 16
OUT_CHANNELS = 32
KP1 = NUM_NEIGHBORS + 1
R = 256
BIG = 3.0e38


def _knn_kernel(rows_ref, ptsT_ref, ptsP_ref, out_ref, scratch_ref):
    rows = rows_ref[0]
    ptsT = ptsT_ref[0]
    ptsP = ptsP_ref[0]
    n = ptsT.shape[1]
    sqr = jnp.sum(rows * rows, axis=1, keepdims=True)
    sqc = jnp.sum(ptsT * ptsT, axis=0, keepdims=True)
    dot = jnp.dot(rows, ptsT, preferred_element_type=jnp.float32)
    dist = (sqr + sqc) - 2.0 * dot
    iota = jax.lax.broadcasted_iota(jnp.int32, (R, n), 1)

    def body(k, d):
        minv = jnp.min(d, axis=1, keepdims=True)
        hit = d == minv
        idxv = jnp.min(jnp.where(hit, iota, n), axis=1, keepdims=True)
        sel = iota == idxv
        coords = jax.lax.dot_general(
            sel.astype(jnp.float32), ptsP,
            (((1,), (0,)), ((), ())),
            precision=jax.lax.Precision.HIGHEST,
            preferred_element_type=jnp.float32)
        scratch_ref[k] = coords
        return jnp.where(sel, BIG, d)

    jax.lax.fori_loop(0, KP1, body, dist, unroll=False)
    out_ref[0] = scratch_ref[...]


def _knn_coords(pts):
    Bb, Nn, _ = pts.shape
    pts_pad = jnp.pad(pts, ((0, 0), (0, 0), (0, 5)))
    ptsT = jnp.transpose(pts_pad, (0, 2, 1))
    nb = Nn // R
    return pl.pallas_call(
        _knn_kernel,
        grid=(Bb, nb),
        in_specs=[
            pl.BlockSpec((1, R, 8), lambda b, i: (b, i, 0)),
            pl.BlockSpec((1, 8, Nn), lambda b, i: (b, 0, 0)),
            pl.BlockSpec((1, Nn, 8), lambda b, i: (b, 0, 0)),
        ],
        out_specs=pl.BlockSpec((1, KP1, R, 8), lambda b, i: (b * nb + i, 0, 0, 0)),
        out_shape=jax.ShapeDtypeStruct((Bb * nb, KP1, R, 8), jnp.float32),
        scratch_shapes=[pltpu.VMEM((KP1, R, 8), jnp.float32)],
    )(pts_pad, ptsT, pts_pad)


CAND = 128
MAXI = 2147483647
NEG = -3.0e38


def _sc_knn_coords(pts):
    Bb, Nn, _ = pts.shape
    nb = Nn // R
    M = Bb * nb
    wpb = 32 // Bb
    bpw = nb // wpb
    ptsx = pts[:, :, 0]
    ptsy = pts[:, :, 1]
    ptsz = pts[:, :, 2]
    f32, i32 = jnp.float32, jnp.int32
    mesh = plsc.VectorSubcoreMesh(core_axis_name="c", subcore_axis_name="s")

    @functools.partial(
        pl.kernel,
        out_type=jax.ShapeDtypeStruct((M, KP1, R, 4), f32),
        mesh=mesh,
        compiler_params=pltpu.CompilerParams(
            needs_layout_passes=False, use_tc_tiling_on_sc=False),
        scratch_types=[
            pltpu.VMEM((Nn,), f32),
            pltpu.VMEM((Nn,), f32),
            pltpu.VMEM((Nn,), f32),
            pltpu.VMEM((Nn,), f32),
            pltpu.VMEM((Nn,), f32),
            pltpu.VMEM((Nn,), f32),
            pltpu.VMEM((Nn,), f32),
            pltpu.VMEM((Nn,), f32),
            pltpu.VMEM((32,), f32),
            pltpu.VMEM((CAND + 16,), f32),
            pltpu.VMEM((CAND + 16,), i32),
            pltpu.VMEM((48,), i32),
            pltpu.VMEM((KP1, R, 4), f32),
        ],
    )
    def sc_kernel(x_hbm, y_hbm, z_hbm, out_hbm,
                  xb, yb, zb, xr, yr, zr, sqb, db, gmb, cdb, cib, sib, ob):
        cid = lax.axis_index("c")
        sid = lax.axis_index("s")
        w = sid * 2 + cid
        b = w // wpb
        blk0 = (w % wpb) * bpw
        pltpu.sync_copy(x_hbm.at[b], xb)
        pltpu.sync_copy(y_hbm.at[b], yb)
        pltpu.sync_copy(z_hbm.at[b], zb)
        lanes = lax.iota(i32, 16)
        lane0 = lanes == 0
        big = jnp.full((16,), BIG, f32)

        def _rnd_bf16(x):
            u = plsc.bitcast(x, i32)
            u = (u + 0x7FFF + ((u >> 16) & 1)) & ~0xFFFF
            return plsc.bitcast(u, f32)

        def prep(j, _):
            sl = pl.ds(j * 16, 16)
            x = xb[sl]
            y = yb[sl]
            z = zb[sl]
            xr[sl] = _rnd_bf16(x)
            yr[sl] = _rnd_bf16(y)
            zr[sl] = _rnd_bf16(z)
            sqb[sl] = (x * x + y * y) + z * z
            return 0
        lax.fori_loop(0, Nn // 16, prep, 0)

        def per_query(i, blk_base):
            qi = blk_base + i
            qiv = jnp.full((16,), qi, i32)
            qx2 = plsc.load_gather(xr, [qiv]) * 2.0
            qy2 = plsc.load_gather(yr, [qiv]) * 2.0
            qz2 = plsc.load_gather(zr, [qiv]) * 2.0

            def gbody(g, _):
                def inner(jj, acc):
                    sl = pl.ds(jj, 16)
                    dot2 = (qx2 * xr[sl] + qy2 * yr[sl]) + qz2 * zr[sl]
                    d = sqb[sl] - dot2
                    db[sl] = d
                    return jnp.minimum(acc, d)
                acc = plsc.parallel_loop(
                    g * 128, g * 128 + 128, 16, unroll=8, carry=big)(inner)
                plsc.store_scatter(gmb, [jnp.full((16,), g, i32)],
                                   jnp.full((16,), jnp.min(acc), f32),
                                   mask=lane0)
                return 0
            lax.fori_loop(0, 32, gbody, 0)

            def tbody(k, carry):
                g0, g1, _ = carry
                m = jnp.min(jnp.minimum(g0, g1))
                return (jnp.where(g0 == m, BIG, g0),
                        jnp.where(g1 == m, BIG, g1), m)
            _, _, T = lax.fori_loop(
                0, KP1, tbody,
                (gmb[pl.ds(0, 16)], gmb[pl.ds(16, 16)], 0.0))

            negv = jnp.full((16,), NEG, f32)
            for v in range(CAND // 16 + 1):
                cdb[pl.ds(v * 16, 16)] = negv

            def cgroup(g, off):
                def do(off):
                    def inner(j, off):
                        d = db[pl.ds(j, 16)]
                        msk = d <= T
                        plsc.store_compressed(
                            cdb.at[pl.ds(off, 16)], d, mask=msk)
                        plsc.store_compressed(
                            cib.at[pl.ds(off, 16)], lanes + j, mask=msk)
                        cnt = jnp.max(plsc.all_reduce_population_count(msk))
                        return jnp.minimum(off + cnt, CAND)
                    return plsc.parallel_loop(
                        g * 128, g * 128 + 128, 16, unroll=8,
                        carry=off)(inner)
                gm = jnp.min(plsc.load_gather(gmb, [jnp.full((16,), g, i32)]))
                return lax.cond(gm <= T, do, lambda o: o, off)
            cnt = lax.fori_loop(0, 32, cgroup, 0)
            nv = (cnt + 15) // 16

            def cull(_, c):
                def vmax(v, mm):
                    return jnp.maximum(mm, jnp.max(cdb[pl.ds(v * 16, 16)]))
                m = lax.fori_loop(0, nv, vmax, NEG)

                def vidx(v, ii):
                    d = cdb[pl.ds(v * 16, 16)]
                    return jnp.maximum(ii, jnp.max(
                        jnp.where(d == m, cib[pl.ds(v * 16, 16)], -1)))
                imax = lax.fori_loop(0, nv, vidx, -1)

                def vclr(v, _):
                    sl = pl.ds(v * 16, 16)
                    d = cdb[sl]
                    hit = (d == m) & (cib[sl] == imax)
                    cdb[sl] = jnp.where(hit, NEG, d)
                    return 0
                lax.fori_loop(0, nv, vclr, 0)
                return 0
            lax.fori_loop(0, cnt - KP1, cull, 0)

            def vmin(v, mm):
                d = cdb[pl.ds(v * 16, 16)]
                return jnp.minimum(mm, jnp.min(jnp.where(d > NEG, d, BIG)))
            m0 = lax.fori_loop(0, nv, vmin, BIG)

            def vi0(v, ii):
                d = cdb[pl.ds(v * 16, 16)]
                return jnp.minimum(ii, jnp.min(
                    jnp.where(d == m0, cib[pl.ds(v * 16, 16)], MAXI)))
            i0 = lax.fori_loop(0, nv, vi0, MAXI)

            def coll(v, off2):
                sl = pl.ds(v * 16, 16)
                d = cdb[sl]
                ci = cib[sl]
                msk = (d > NEG) & ((d != m0) | (ci != i0))
                plsc.store_compressed(sib.at[pl.ds(off2, 16)], ci, mask=msk)
                return off2 + jnp.max(plsc.all_reduce_population_count(msk))
            lax.fori_loop(0, nv, coll, 0)

            si = sib[pl.ds(0, 16)]
            i0v = jnp.full((16,), i0, i32)
            ivec = jnp.full((16,), i, i32)
            zv = jnp.zeros((16,), i32)
            for coord, buf in ((0, xb), (1, yb), (2, zb)):
                cvec = jnp.full((16,), coord, i32)
                v1 = plsc.load_gather(buf, [si])
                v0 = plsc.load_gather(buf, [i0v])
                plsc.store_scatter(ob, [lanes + 1, ivec, cvec], v1)
                plsc.store_scatter(ob, [zv, ivec, cvec], v0, mask=lane0)
            return blk_base

        for blk in range(bpw):
            mloc = blk0 + blk
            lax.fori_loop(0, R, per_query, mloc * R)
            pltpu.sync_copy(ob, out_hbm.at[b * nb + mloc])

    return sc_kernel(ptsx, ptsy, ptsz)


def _bn_scale_shift(s, ss, cnt, g, b, eps=1e-5):
    mean = s / cnt
    var = ss / cnt - mean * mean
    scale = g / jnp.sqrt(var + eps)
    shift = b - mean * scale
    return scale.reshape(1, -1), shift.reshape(1, -1)


def _acc_init(i0_ref, i1_ref):
    @pl.when(pl.program_id(0) == 0)
    def _():
        i0_ref[...] = jnp.zeros_like(i0_ref)
        i1_ref[...] = jnp.zeros_like(i1_ref)


def _feat_y1_kernel(c_ref, w_ref, y1_ref, s_ref, ss_ref):
    c = c_ref[0][:, :, :3]
    ab = c[:1]
    rel = c[1:] - ab
    d = jnp.sqrt(jnp.sum(rel * rel, axis=2, keepdims=True) + 1e-8)
    feat = jnp.concatenate(
        (jnp.broadcast_to(ab, rel.shape), rel, d,
         jnp.zeros_like(d)), axis=2)
    y1 = jnp.dot(feat.reshape(16 * R, 8), w_ref[...],
                 preferred_element_type=jnp.float32)
    y1_ref[0] = y1.reshape(16, R, 16)
    _acc_init(s_ref, ss_ref)
    s_ref[...] += jnp.sum(y1, axis=0, keepdims=True)
    ss_ref[...] += jnp.sum(y1 * y1, axis=0, keepdims=True)


def _y2_kernel(y1_ref, w_ref, sc_ref, sh_ref, y2_ref, s_ref, ss_ref):
    h1 = jnp.maximum(y1_ref[0].reshape(16 * R, 16) * sc_ref[...]
                     + sh_ref[...], 0.0)
    y2 = jnp.dot(h1, w_ref[...], preferred_element_type=jnp.float32)
    y2_ref[0] = y2.reshape(16, R, 32)
    _acc_init(s_ref, ss_ref)
    s_ref[...] += jnp.sum(y2, axis=0, keepdims=True)
    ss_ref[...] += jnp.sum(y2 * y2, axis=0, keepdims=True)


def _pool_lift_kernel(y2_ref, pts_ref, w2_ref, sc_ref, sh_ref,
                      pool_ref, y3_ref, s_ref, ss_ref):
    h2 = jnp.maximum(y2_ref[0] * sc_ref[...].reshape(1, 1, 32)
                     + sh_ref[...].reshape(1, 1, 32), 0.0)
    pool_ref[0] = jnp.max(h2, axis=0)
    y3 = jnp.dot(pts_ref[0], w2_ref[...],
                 preferred_element_type=jnp.float32)
    y3_ref[0] = y3
    _acc_init(s_ref, ss_ref)
    s_ref[...] += jnp.sum(y3, axis=0, keepdims=True)
    ss_ref[...] += jnp.sum(y3 * y3, axis=0, keepdims=True)


def _y4_kernel(y3_ref, pool_ref, w3_ref, sc_ref, sh_ref,
               y4_ref, s_ref, ss_ref):
    lifted = jnp.maximum(y3_ref[0] * sc_ref[...] + sh_ref[...], 0.0)
    xcat = jnp.concatenate((lifted, pool_ref[0]), axis=1)
    y4 = jnp.dot(xcat, w3_ref[...], preferred_element_type=jnp.float32)
    y4_ref[0] = y4
    _acc_init(s_ref, ss_ref)
    s_ref[...] += jnp.sum(y4, axis=0, keepdims=True)
    ss_ref[...] += jnp.sum(y4 * y4, axis=0, keepdims=True)


def _final_kernel(y4_ref, sc_ref, sh_ref, o_ref):
    o_ref[0] = jnp.maximum(y4_ref[0] * sc_ref[...] + sh_ref[...], 0.0)


def _full_spec(*shape):
    nd = len(shape)
    return pl.BlockSpec(shape, lambda m: (0,) * nd)


def _mblk_spec(*rest):
    return pl.BlockSpec((1,) + rest, lambda m: (m,) + (0,) * len(rest))


def kernel(pts, W1a, g1a, b1a, W1b, g1b, b1b, W2, g2, b2, W3, g3, b3):
    Bb, Nn, _ = pts.shape
    nb = Nn // R
    M = Bb * nb
    cnt_nbr = float(Bb * Nn * 16)
    cnt_pts = float(Bb * Nn)
    coords = _sc_knn_coords(pts)
    pts_pad = jnp.pad(pts, ((0, 0), (0, 0), (0, 5))).reshape(M, R, 8)
    W1a_p = jnp.pad(W1a, ((0, 1), (0, 0)))
    W2_p = jnp.pad(W2, ((0, 5), (0, 0)))

    f32 = jnp.float32
    y1, s1, ss1 = pl.pallas_call(
        _feat_y1_kernel, grid=(M,),
        in_specs=[_mblk_spec(KP1, R, 4), _full_spec(8, 16)],
        out_specs=[_mblk_spec(16, R, 16), _full_spec(1, 16), _full_spec(1, 16)],
        out_shape=[jax.ShapeDtypeStruct((M, 16, R, 16), f32),
                   jax.ShapeDtypeStruct((1, 16), f32),
                   jax.ShapeDtypeStruct((1, 16), f32)],
    )(coords, W1a_p)
    sc1, sh1 = _bn_scale_shift(s1[0], ss1[0], cnt_nbr, g1a, b1a)

    y2, s2, ss2 = pl.pallas_call(
        _y2_kernel, grid=(M,),
        in_specs=[_mblk_spec(16, R, 16), _full_spec(16, 32),
                  _full_spec(1, 16), _full_spec(1, 16)],
        out_specs=[_mblk_spec(16, R, 32), _full_spec(1, 32), _full_spec(1, 32)],
        out_shape=[jax.ShapeDtypeStruct((M, 16, R, 32), f32),
                   jax.ShapeDtypeStruct((1, 32), f32),
                   jax.ShapeDtypeStruct((1, 32), f32)],
    )(y1, W1b, sc1, sh1)
    sc2, sh2 = _bn_scale_shift(s2[0], ss2[0], cnt_nbr, g1b, b1b)

    pooled, y3, s3, ss3 = pl.pallas_call(
        _pool_lift_kernel, grid=(M,),
        in_specs=[_mblk_spec(16, R, 32), _mblk_spec(R, 8),
                  _full_spec(8, 32), _full_spec(1, 32), _full_spec(1, 32)],
        out_specs=[_mblk_spec(R, 32), _mblk_spec(R, 32),
                   _full_spec(1, 32), _full_spec(1, 32)],
        out_shape=[jax.ShapeDtypeStruct((M, R, 32), f32),
                   jax.ShapeDtypeStruct((M, R, 32), f32),
                   jax.ShapeDtypeStruct((1, 32), f32),
                   jax.ShapeDtypeStruct((1, 32), f32)],
    )(y2, pts_pad, W2_p, sc2, sh2)
    sc3, sh3 = _bn_scale_shift(s3[0], ss3[0], cnt_pts, g2, b2)

    y4, s4, ss4 = pl.pallas_call(
        _y4_kernel, grid=(M,),
        in_specs=[_mblk_spec(R, 32), _mblk_spec(R, 32), _full_spec(64, 32),
                  _full_spec(1, 32), _full_spec(1, 32)],
        out_specs=[_mblk_spec(R, 32), _full_spec(1, 32), _full_spec(1, 32)],
        out_shape=[jax.ShapeDtypeStruct((M, R, 32), f32),
                   jax.ShapeDtypeStruct((1, 32), f32),
                   jax.ShapeDtypeStruct((1, 32), f32)],
    )(y3, pooled, W3, sc3, sh3)
    sc4, sh4 = _bn_scale_shift(s4[0], ss4[0], cnt_pts, g3, b3)

    out = pl.pallas_call(
        _final_kernel, grid=(M,),
        in_specs=[_mblk_spec(R, 32), _full_spec(1, 32), _full_spec(1, 32)],
        out_specs=_mblk_spec(R, 32),
        out_shape=jax.ShapeDtypeStruct((M, R, 32), f32),
    )(y4, sc4, sh4)
    return out.reshape(Bb, Nn, OUT_CHANNELS)


# speedup vs baseline: 3.1142x; 1.0019x over previous
import functools

import jax
import jax.numpy as jnp
from jax import lax
from jax.experimental import pallas as pl
from jax.experimental.pallas import tpu as pltpu
from jax.experimental.pallas import tpu_sc as plsc

NUM_NEIGHBORS = 16
OUT_CHANNELS = 32
KP1 = NUM_NEIGHBORS + 1
R = 256
BIG = 3.0e38


def _knn_kernel(rows_ref, ptsT_ref, ptsP_ref, out_ref, scratch_ref):
    rows = rows_ref[0]
    ptsT = ptsT_ref[0]
    ptsP = ptsP_ref[0]
    n = ptsT.shape[1]
    sqr = jnp.sum(rows * rows, axis=1, keepdims=True)
    sqc = jnp.sum(ptsT * ptsT, axis=0, keepdims=True)
    dot = jnp.dot(rows, ptsT, preferred_element_type=jnp.float32)
    dist = (sqr + sqc) - 2.0 * dot
    iota = jax.lax.broadcasted_iota(jnp.int32, (R, n), 1)

    def body(k, d):
        minv = jnp.min(d, axis=1, keepdims=True)
        hit = d == minv
        idxv = jnp.min(jnp.where(hit, iota, n), axis=1, keepdims=True)
        sel = iota == idxv
        coords = jax.lax.dot_general(
            sel.astype(jnp.float32), ptsP,
            (((1,), (0,)), ((), ())),
            precision=jax.lax.Precision.HIGHEST,
            preferred_element_type=jnp.float32)
        scratch_ref[k] = coords
        return jnp.where(sel, BIG, d)

    jax.lax.fori_loop(0, KP1, body, dist, unroll=False)
    out_ref[0] = scratch_ref[...]


def _knn_coords(pts):
    Bb, Nn, _ = pts.shape
    pts_pad = jnp.pad(pts, ((0, 0), (0, 0), (0, 5)))
    ptsT = jnp.transpose(pts_pad, (0, 2, 1))
    nb = Nn // R
    return pl.pallas_call(
        _knn_kernel,
        grid=(Bb, nb),
        in_specs=[
            pl.BlockSpec((1, R, 8), lambda b, i: (b, i, 0)),
            pl.BlockSpec((1, 8, Nn), lambda b, i: (b, 0, 0)),
            pl.BlockSpec((1, Nn, 8), lambda b, i: (b, 0, 0)),
        ],
        out_specs=pl.BlockSpec((1, KP1, R, 8), lambda b, i: (b * nb + i, 0, 0, 0)),
        out_shape=jax.ShapeDtypeStruct((Bb * nb, KP1, R, 8), jnp.float32),
        scratch_shapes=[pltpu.VMEM((KP1, R, 8), jnp.float32)],
    )(pts_pad, ptsT, pts_pad)


CAND = 128
MAXI = 2147483647
NEG = -3.0e38


def _sc_knn_coords(pts):
    Bb, Nn, _ = pts.shape
    nb = Nn // R
    M = Bb * nb
    wpb = 32 // Bb
    bpw = nb // wpb
    ptsx = pts[:, :, 0]
    ptsy = pts[:, :, 1]
    ptsz = pts[:, :, 2]
    f32, i32 = jnp.float32, jnp.int32
    mesh = plsc.VectorSubcoreMesh(core_axis_name="c", subcore_axis_name="s")

    @functools.partial(
        pl.kernel,
        out_type=jax.ShapeDtypeStruct((M, KP1, R, 4), f32),
        mesh=mesh,
        compiler_params=pltpu.CompilerParams(
            needs_layout_passes=False, use_tc_tiling_on_sc=False),
        scratch_types=[
            pltpu.VMEM((Nn,), f32),
            pltpu.VMEM((Nn,), f32),
            pltpu.VMEM((Nn,), f32),
            pltpu.VMEM((Nn,), f32),
            pltpu.VMEM((Nn,), f32),
            pltpu.VMEM((Nn,), f32),
            pltpu.VMEM((Nn,), f32),
            pltpu.VMEM((Nn,), f32),
            pltpu.VMEM((32,), f32),
            pltpu.VMEM((CAND + 16,), f32),
            pltpu.VMEM((CAND + 16,), i32),
            pltpu.VMEM((48,), i32),
            pltpu.VMEM((KP1, R, 4), f32),
        ],
    )
    def sc_kernel(x_hbm, y_hbm, z_hbm, out_hbm,
                  xb, yb, zb, xr, yr, zr, sqb, db, gmb, cdb, cib, sib, ob):
        cid = lax.axis_index("c")
        sid = lax.axis_index("s")
        w = sid * 2 + cid
        b = w // wpb
        blk0 = (w % wpb) * bpw
        pltpu.sync_copy(x_hbm.at[b], xb)
        pltpu.sync_copy(y_hbm.at[b], yb)
        pltpu.sync_copy(z_hbm.at[b], zb)
        lanes = lax.iota(i32, 16)
        lane0 = lanes == 0
        big = jnp.full((16,), BIG, f32)

        def _rnd_bf16(x):
            u = plsc.bitcast(x, i32)
            u = (u + 0x7FFF + ((u >> 16) & 1)) & ~0xFFFF
            return plsc.bitcast(u, f32)

        def prep(j, _):
            sl = pl.ds(j * 16, 16)
            x = xb[sl]
            y = yb[sl]
            z = zb[sl]
            xr[sl] = _rnd_bf16(x)
            yr[sl] = _rnd_bf16(y)
            zr[sl] = _rnd_bf16(z)
            sqb[sl] = (x * x + y * y) + z * z
            return 0
        lax.fori_loop(0, Nn // 16, prep, 0)

        def per_query(i, blk_base):
            qi = blk_base + i
            qiv = jnp.full((16,), qi, i32)
            qx2 = plsc.load_gather(xr, [qiv]) * 2.0
            qy2 = plsc.load_gather(yr, [qiv]) * 2.0
            qz2 = plsc.load_gather(zr, [qiv]) * 2.0

            def gbody(g, _):
                def inner(jj, acc):
                    sl = pl.ds(jj, 16)
                    dot2 = (qx2 * xr[sl] + qy2 * yr[sl]) + qz2 * zr[sl]
                    d = sqb[sl] - dot2
                    db[sl] = d
                    return jnp.minimum(acc, d)
                acc = plsc.parallel_loop(
                    g * 128, g * 128 + 128, 16, unroll=8, carry=big)(inner)
                plsc.store_scatter(gmb, [jnp.full((16,), g, i32)],
                                   jnp.full((16,), jnp.min(acc), f32),
                                   mask=lane0)
                return 0
            lax.fori_loop(0, 32, gbody, 0)

            def tbody(k, carry):
                g0, g1, _ = carry
                m = jnp.min(jnp.minimum(g0, g1))
                return (jnp.where(g0 == m, BIG, g0),
                        jnp.where(g1 == m, BIG, g1), m)
            _, _, T = lax.fori_loop(
                0, KP1, tbody,
                (gmb[pl.ds(0, 16)], gmb[pl.ds(16, 16)], 0.0))

            negv = jnp.full((16,), NEG, f32)
            for v in range(CAND // 16 + 1):
                cdb[pl.ds(v * 16, 16)] = negv

            def cgroup(g, off):
                def do(off):
                    def inner(j, off):
                        d = db[pl.ds(j, 16)]
                        msk = d <= T
                        plsc.store_compressed(
                            cdb.at[pl.ds(off, 16)], d, mask=msk)
                        plsc.store_compressed(
                            cib.at[pl.ds(off, 16)], lanes + j, mask=msk)
                        cnt = jnp.max(plsc.all_reduce_population_count(msk))
                        return jnp.minimum(off + cnt, CAND)
                    return plsc.parallel_loop(
                        g * 128, g * 128 + 128, 16, unroll=8,
                        carry=off)(inner)
                gm = jnp.min(plsc.load_gather(gmb, [jnp.full((16,), g, i32)]))
                return lax.cond(gm <= T, do, lambda o: o, off)
            cnt = lax.fori_loop(0, 32, cgroup, 0)
            nv = (cnt + 15) // 16

            def cull(_, c):
                def vmax(v, mm):
                    return jnp.maximum(mm, jnp.max(cdb[pl.ds(v * 16, 16)]))
                m = lax.fori_loop(0, nv, vmax, NEG)

                def vidx(v, ii):
                    d = cdb[pl.ds(v * 16, 16)]
                    return jnp.maximum(ii, jnp.max(
                        jnp.where(d == m, cib[pl.ds(v * 16, 16)], -1)))
                imax = lax.fori_loop(0, nv, vidx, -1)

                def vclr(v, _):
                    sl = pl.ds(v * 16, 16)
                    d = cdb[sl]
                    hit = (d == m) & (cib[sl] == imax)
                    cdb[sl] = jnp.where(hit, NEG, d)
                    return 0
                lax.fori_loop(0, nv, vclr, 0)
                return 0
            lax.fori_loop(0, cnt - KP1, cull, 0)

            def vmin(v, mm):
                d = cdb[pl.ds(v * 16, 16)]
                return jnp.minimum(mm, jnp.min(jnp.where(d > NEG, d, BIG)))
            m0 = lax.fori_loop(0, nv, vmin, BIG)

            def vi0(v, ii):
                d = cdb[pl.ds(v * 16, 16)]
                return jnp.minimum(ii, jnp.min(
                    jnp.where(d == m0, cib[pl.ds(v * 16, 16)], MAXI)))
            i0 = lax.fori_loop(0, nv, vi0, MAXI)

            def coll(v, off2):
                sl = pl.ds(v * 16, 16)
                d = cdb[sl]
                ci = cib[sl]
                msk = (d > NEG) & ((d != m0) | (ci != i0))
                plsc.store_compressed(sib.at[pl.ds(off2, 16)], ci, mask=msk)
                return off2 + jnp.max(plsc.all_reduce_population_count(msk))
            lax.fori_loop(0, nv, coll, 0)

            si = sib[pl.ds(0, 16)]
            i0v = jnp.full((16,), i0, i32)
            ivec = jnp.full((16,), i, i32)
            zv = jnp.zeros((16,), i32)
            for coord, buf in ((0, xb), (1, yb), (2, zb)):
                cvec = jnp.full((16,), coord, i32)
                v1 = plsc.load_gather(buf, [si])
                v0 = plsc.load_gather(buf, [i0v])
                plsc.store_scatter(ob, [lanes + 1, ivec, cvec], v1)
                plsc.store_scatter(ob, [zv, ivec, cvec], v0, mask=lane0)
            return blk_base

        for blk in range(bpw):
            mloc = blk0 + blk
            lax.fori_loop(0, R, per_query, mloc * R)
            pltpu.sync_copy(ob, out_hbm.at[b * nb + mloc])

    return sc_kernel(ptsx, ptsy, ptsz)


def _bn_in(s_ref, ss_ref, g_ref, b_ref, cnt, eps=1e-5):
    mean = s_ref[...] / cnt
    var = ss_ref[...] / cnt - mean * mean
    scale = g_ref[...] / jnp.sqrt(var + eps)
    shift = b_ref[...] - mean * scale
    return scale, shift


def _acc_init(i0_ref, i1_ref):
    @pl.when(pl.program_id(0) == 0)
    def _():
        i0_ref[...] = jnp.zeros_like(i0_ref)
        i1_ref[...] = jnp.zeros_like(i1_ref)


def _feat_y1_kernel(c_ref, w_ref, y1_ref, s_ref, ss_ref):
    c = c_ref[0][:, :, :3]
    ab = c[:1]
    rel = c[1:] - ab
    d = jnp.sqrt(jnp.sum(rel * rel, axis=2, keepdims=True) + 1e-8)
    feat = jnp.concatenate(
        (jnp.broadcast_to(ab, rel.shape), rel, d,
         jnp.zeros_like(d)), axis=2)
    y1 = jnp.dot(feat.reshape(16 * R, 8), w_ref[...],
                 preferred_element_type=jnp.float32)
    y1_ref[0] = y1.reshape(16, R, 16)
    _acc_init(s_ref, ss_ref)
    s_ref[...] += jnp.sum(y1, axis=0, keepdims=True)
    ss_ref[...] += jnp.sum(y1 * y1, axis=0, keepdims=True)


def _y2_kernel(cnt, y1_ref, w_ref, s1_ref, ss1_ref, g_ref, b_ref,
               y2_ref, s_ref, ss_ref):
    sc, sh = _bn_in(s1_ref, ss1_ref, g_ref, b_ref, cnt)
    h1 = jnp.maximum(y1_ref[0].reshape(16 * R, 16) * sc + sh, 0.0)
    y2 = jnp.dot(h1, w_ref[...], preferred_element_type=jnp.float32)
    y2_ref[0] = y2.reshape(16, R, 32)
    _acc_init(s_ref, ss_ref)
    s_ref[...] += jnp.sum(y2, axis=0, keepdims=True)
    ss_ref[...] += jnp.sum(y2 * y2, axis=0, keepdims=True)


def _pool_lift_kernel(cnt, y2_ref, pts_ref, w2_ref, s2_ref, ss2_ref,
                      g_ref, b_ref, pool_ref, y3_ref, s_ref, ss_ref):
    sc, sh = _bn_in(s2_ref, ss2_ref, g_ref, b_ref, cnt)
    h2 = jnp.maximum(y2_ref[0] * sc.reshape(1, 1, 32)
                     + sh.reshape(1, 1, 32), 0.0)
    pool_ref[0] = jnp.max(h2, axis=0)
    y3 = jnp.dot(pts_ref[0], w2_ref[...],
                 preferred_element_type=jnp.float32)
    y3_ref[0] = y3
    _acc_init(s_ref, ss_ref)
    s_ref[...] += jnp.sum(y3, axis=0, keepdims=True)
    ss_ref[...] += jnp.sum(y3 * y3, axis=0, keepdims=True)


def _y4_kernel(cnt, y3_ref, pool_ref, w3_ref, s3_ref, ss3_ref,
               g_ref, b_ref, y4_ref, s_ref, ss_ref):
    sc, sh = _bn_in(s3_ref, ss3_ref, g_ref, b_ref, cnt)
    lifted = jnp.maximum(y3_ref[0] * sc + sh, 0.0)
    xcat = jnp.concatenate((lifted, pool_ref[0]), axis=1)
    y4 = jnp.dot(xcat, w3_ref[...], preferred_element_type=jnp.float32)
    y4_ref[0] = y4
    _acc_init(s_ref, ss_ref)
    s_ref[...] += jnp.sum(y4, axis=0, keepdims=True)
    ss_ref[...] += jnp.sum(y4 * y4, axis=0, keepdims=True)


def _final_kernel(cnt, y4_ref, s4_ref, ss4_ref, g_ref, b_ref, o_ref):
    sc, sh = _bn_in(s4_ref, ss4_ref, g_ref, b_ref, cnt)
    o_ref[0] = jnp.maximum(y4_ref[0] * sc + sh, 0.0)


def _full_spec(*shape):
    nd = len(shape)
    return pl.BlockSpec(shape, lambda m: (0,) * nd)


def _mblk_spec(*rest):
    return pl.BlockSpec((1,) + rest, lambda m: (m,) + (0,) * len(rest))


def kernel(pts, W1a, g1a, b1a, W1b, g1b, b1b, W2, g2, b2, W3, g3, b3):
    Bb, Nn, _ = pts.shape
    nb = Nn // R
    M = Bb * nb
    cnt_nbr = float(Bb * Nn * 16)
    cnt_pts = float(Bb * Nn)
    coords = _sc_knn_coords(pts)
    pts_pad = jnp.pad(pts, ((0, 0), (0, 0), (0, 5))).reshape(M, R, 8)
    W1a_p = jnp.pad(W1a, ((0, 1), (0, 0)))
    W2_p = jnp.pad(W2, ((0, 5), (0, 0)))

    f32 = jnp.float32
    y1, s1, ss1 = pl.pallas_call(
        _feat_y1_kernel, grid=(M,),
        in_specs=[_mblk_spec(KP1, R, 4), _full_spec(8, 16)],
        out_specs=[_mblk_spec(16, R, 16), _full_spec(1, 16), _full_spec(1, 16)],
        out_shape=[jax.ShapeDtypeStruct((M, 16, R, 16), f32),
                   jax.ShapeDtypeStruct((1, 16), f32),
                   jax.ShapeDtypeStruct((1, 16), f32)],
    )(coords, W1a_p)

    y2, s2, ss2 = pl.pallas_call(
        functools.partial(_y2_kernel, cnt_nbr), grid=(M,),
        in_specs=[_mblk_spec(16, R, 16), _full_spec(16, 32),
                  _full_spec(1, 16), _full_spec(1, 16),
                  _full_spec(1, 16), _full_spec(1, 16)],
        out_specs=[_mblk_spec(16, R, 32), _full_spec(1, 32), _full_spec(1, 32)],
        out_shape=[jax.ShapeDtypeStruct((M, 16, R, 32), f32),
                   jax.ShapeDtypeStruct((1, 32), f32),
                   jax.ShapeDtypeStruct((1, 32), f32)],
    )(y1, W1b, s1, ss1, g1a.reshape(1, -1), b1a.reshape(1, -1))

    pooled, y3, s3, ss3 = pl.pallas_call(
        functools.partial(_pool_lift_kernel, cnt_nbr), grid=(M,),
        in_specs=[_mblk_spec(16, R, 32), _mblk_spec(R, 8), _full_spec(8, 32),
                  _full_spec(1, 32), _full_spec(1, 32),
                  _full_spec(1, 32), _full_spec(1, 32)],
        out_specs=[_mblk_spec(R, 32), _mblk_spec(R, 32),
                   _full_spec(1, 32), _full_spec(1, 32)],
        out_shape=[jax.ShapeDtypeStruct((M, R, 32), f32),
                   jax.ShapeDtypeStruct((M, R, 32), f32),
                   jax.ShapeDtypeStruct((1, 32), f32),
                   jax.ShapeDtypeStruct((1, 32), f32)],
    )(y2, pts_pad, W2_p, s2, ss2, g1b.reshape(1, -1), b1b.reshape(1, -1))

    y4, s4, ss4 = pl.pallas_call(
        functools.partial(_y4_kernel, cnt_pts), grid=(M,),
        in_specs=[_mblk_spec(R, 32), _mblk_spec(R, 32), _full_spec(64, 32),
                  _full_spec(1, 32), _full_spec(1, 32),
                  _full_spec(1, 32), _full_spec(1, 32)],
        out_specs=[_mblk_spec(R, 32), _full_spec(1, 32), _full_spec(1, 32)],
        out_shape=[jax.ShapeDtypeStruct((M, R, 32), f32),
                   jax.ShapeDtypeStruct((1, 32), f32),
                   jax.ShapeDtypeStruct((1, 32), f32)],
    )(y3, pooled, W3, s3, ss3, g2.reshape(1, -1), b2.reshape(1, -1))

    out = pl.pallas_call(
        functools.partial(_final_kernel, cnt_pts), grid=(M,),
        in_specs=[_mblk_spec(R, 32), _full_spec(1, 32), _full_spec(1, 32),
                  _full_spec(1, 32), _full_spec(1, 32)],
        out_specs=_mblk_spec(R, 32),
        out_shape=jax.ShapeDtypeStruct((M, R, 32), f32),
    )(y4, s4, ss4, g3.reshape(1, -1), b3.reshape(1, -1))
    return out.reshape(Bb, Nn, OUT_CHANNELS)
